# Initial kernel scaffold; baseline (speedup 1.0000x reference)
import jax
import jax.numpy as jnp
from jax.experimental import pallas as pl


def kernel(x, vec, edge_index, r_ij, f_ij, d_ij, Wvec, Wq, bq, Wk, bk, Wv, bv, Wdk, bdk, Wdv, bdv, Ws, bs, Wf, bf, Wwsrc, Wwtrg, Wtsrc, Wttrg, Wo, bo):
    raise NotImplementedError("write your pallas kernel here")


# speedup vs baseline: 1.0426x; 1.0426x over previous
import functools
import jax
import jax.numpy as jnp
from jax.experimental import pallas as pl

N = 10000
E = 160000
H = 128
HEADS = 8
HD = H // HEADS
CUTOFF = 5.0

NB = 1000
EB = 1600


def _silu(x):
    return x * jax.nn.sigmoid(x)


def _k1_body(x, v0, v1, v2, Wq, bq, Wk, bk, Wv, bv, Wvec, Wwtrg, Wwsrc,
             Wttrg, Wtsrc,
             q_o, k_o, v_o, vdot_o, TT_o, A0_o, A1_o, A2_o, B0_o, B1_o, B2_o,
             v3a_o, v3b_o, v3c_o):
    xb = x[...]
    q_o[...] = xb @ Wq[...] + bq[...]
    k_o[...] = xb @ Wk[...] + bk[...]
    v_o[...] = xb @ Wv[...] + bv[...]
    vs = (v0[...], v1[...], v2[...])
    Wv3 = Wvec[...]
    vdot = jnp.zeros_like(xb)
    TT = jnp.zeros_like(xb)
    Wt1 = Wttrg[...]
    Wt2 = Wtsrc[...]
    Ww1 = Wwtrg[...]
    Ww2 = Wwsrc[...]
    A_out = (A0_o, A1_o, A2_o)
    B_out = (B0_o, B1_o, B2_o)
    v3_out = (v3a_o, v3b_o, v3c_o)
    for a in range(3):
        va = vs[a]
        v123 = va @ Wv3
        vdot = vdot + v123[:, :H] * v123[:, H:2 * H]
        v3_out[a][...] = v123[:, 2 * H:]
        TT = TT + (va @ Wt1) * (va @ Wt2)
        A_out[a][...] = va @ Ww1
        B_out[a][...] = va @ Ww2
    vdot_o[...] = vdot
    TT_o[...] = TT


def _node_precompute(x, vec, Wq, bq, Wk, bk, Wv, bv, Wvec, Wwtrg, Wwsrc,
                     Wttrg, Wtsrc):
    grid = (N // NB,)
    bn = pl.BlockSpec((NB, H), lambda i: (i, 0))
    bw = lambda s: pl.BlockSpec(s, lambda i: (0, 0))
    outs = [jax.ShapeDtypeStruct((N, H), jnp.float32)] * 14
    return pl.pallas_call(
        _k1_body,
        grid=grid,
        in_specs=[bn, bn, bn, bn,
                  bw((H, H)), bw((1, H)), bw((H, H)), bw((1, H)),
                  bw((H, H)), bw((1, H)), bw((H, 3 * H)),
                  bw((H, H)), bw((H, H)), bw((H, H)), bw((H, H))],
        out_specs=[bn] * 14,
        out_shape=outs,
    )(x, vec[:, 0, :], vec[:, 1, :], vec[:, 2, :],
      Wq, bq.reshape(1, H), Wk, bk.reshape(1, H), Wv, bv.reshape(1, H),
      Wvec, Wwtrg, Wwsrc, Wttrg, Wtsrc)


def _k2_body(f, r, Wdk, bdk, Wdv, bdv, dk_o, dv_o, cut_o):
    fb = f[...]
    dk_o[...] = _silu(fb @ Wdk[...] + bdk[...])
    dv_o[...] = _silu(fb @ Wdv[...] + bdv[...])
    rb = r[...]
    cut_o[...] = jnp.where(rb < CUTOFF,
                           0.5 * (jnp.cos(jnp.pi * rb / CUTOFF) + 1.0), 0.0)


def _edge_dense(f_ij, r_ij, Wdk, bdk, Wdv, bdv):
    grid = (E // EB,)
    be = pl.BlockSpec((EB, H), lambda i: (i, 0))
    b1 = pl.BlockSpec((EB, 1), lambda i: (i, 0))
    bw = lambda s: pl.BlockSpec(s, lambda i: (0, 0))
    return pl.pallas_call(
        _k2_body,
        grid=grid,
        in_specs=[be, b1, bw((H, H)), bw((1, H)), bw((H, H)), bw((1, H))],
        out_specs=[be, be, b1],
        out_shape=[jax.ShapeDtypeStruct((E, H), jnp.float32),
                   jax.ShapeDtypeStruct((E, H), jnp.float32),
                   jax.ShapeDtypeStruct((E, 1), jnp.float32)],
    )(f_ij, r_ij.reshape(E, 1), Wdk, bdk.reshape(1, H), Wdv, bdv.reshape(1, H))


def _k3_body(q_i, k_j, v_j, dk, dv, cut, f, TT_e,
             vj0, vj1, vj2, vi0, vi1, vi2, Ai0, Ai1, Ai2, Bj0, Bj1, Bj2,
             d0, d1, d2, hsum, hbc, Ws, bs, Wf, bf, Wwtrg, Wwsrc, Wttrg, Wtsrc,
             ve_o, m0_o, m1_o, m2_o, df_o):
    qkd = q_i[...] * k_j[...] * dk[...]
    attn = _silu(qkd @ hsum[...]) * cut[...]
    attn_b = attn @ hbc[...]
    v_e = v_j[...] * dv[...] * attn_b
    ve_o[...] = v_e
    s = _silu(v_e @ Ws[...] + bs[...])
    s1, s2 = s[:, :H], s[:, H:]
    vjs = (vj0[...], vj1[...], vj2[...])
    vis = (vi0[...], vi1[...], vi2[...])
    ds = (d0[...], d1[...], d2[...])
    m_out = (m0_o, m1_o, m2_o)
    dvec_i = jnp.zeros_like(s1)
    dvec_j = jnp.zeros_like(s1)
    wterm1 = jnp.zeros_like(s1)
    Ais = (Ai0[...], Ai1[...], Ai2[...])
    Bjs = (Bj0[...], Bj1[...], Bj2[...])
    for a in range(3):
        m_out[a][...] = vjs[a] * s1 + s2 * ds[a]
        dvec_i = dvec_i + ds[a] * vis[a]
        dvec_j = dvec_j + ds[a] * vjs[a]
        wterm1 = wterm1 + Ais[a] * Bjs[a]
    w_dot = wterm1 - (dvec_i @ Wwtrg[...]) * (dvec_j @ Wwsrc[...])
    t_dot = TT_e[...] - (dvec_i @ Wttrg[...]) * (dvec_i @ Wtsrc[...])
    ff = _silu(f[...] @ Wf[...] + bf[...])
    df_o[...] = ff[:, :H] * w_dot + ff[:, H:] * t_dot


def _edge_mega(q_i, k_j, v_j, dk, dv, cut, f_ij, TT_e, vj, vi, Ai, Bj, d_ij,
               Ws, bs, Wf, bf, Wwtrg, Wwsrc, Wttrg, Wtsrc):
    grid = (E // EB,)
    be = pl.BlockSpec((EB, H), lambda i: (i, 0))
    b1 = pl.BlockSpec((EB, 1), lambda i: (i, 0))
    bw = lambda s: pl.BlockSpec(s, lambda i: (0, 0))
    hsum = (jnp.arange(H)[:, None] // HD == jnp.arange(HEADS)[None, :]
            ).astype(jnp.float32)
    hbc = hsum.T
    ins = ([be] * 5 + [b1] + [be] * 2 + [be] * 12 + [b1] * 3 +
           [bw((H, HEADS)), bw((HEADS, H)), bw((H, 2 * H)), bw((1, 2 * H)),
            bw((H, 2 * H)), bw((1, 2 * H)),
            bw((H, H)), bw((H, H)), bw((H, H)), bw((H, H))])
    return pl.pallas_call(
        _k3_body,
        grid=grid,
        in_specs=ins,
        out_specs=[be] * 5,
        out_shape=[jax.ShapeDtypeStruct((E, H), jnp.float32)] * 5,
    )(q_i, k_j, v_j, dk, dv, cut, f_ij, TT_e,
      vj[:, 0, :], vj[:, 1, :], vj[:, 2, :],
      vi[:, 0, :], vi[:, 1, :], vi[:, 2, :],
      Ai[:, 0, :], Ai[:, 1, :], Ai[:, 2, :],
      Bj[:, 0, :], Bj[:, 1, :], Bj[:, 2, :],
      d_ij[:, 0:1], d_ij[:, 1:2], d_ij[:, 2:3],
      hsum, hbc, Ws, bs.reshape(1, 2 * H), Wf, bf.reshape(1, 2 * H),
      Wwtrg, Wwsrc, Wttrg, Wtsrc)


def _k5_body(xa, vdot, v30, v31, v32, g0, g1, g2, Wo, bo,
             dx_o, dv0_o, dv1_o, dv2_o):
    o = xa[...] @ Wo[...] + bo[...]
    o1, o2, o3 = o[:, :H], o[:, H:2 * H], o[:, 2 * H:]
    dx_o[...] = vdot[...] * o2 + o3
    dv0_o[...] = v30[...] * o1 + g0[...]
    dv1_o[...] = v31[...] * o1 + g1[...]
    dv2_o[...] = v32[...] * o1 + g2[...]


def _node_epilogue(x_agg, vec_dot, v3, vec_agg, Wo, bo):
    grid = (N // NB,)
    bn = pl.BlockSpec((NB, H), lambda i: (i, 0))
    bw = lambda s: pl.BlockSpec(s, lambda i: (0, 0))
    return pl.pallas_call(
        _k5_body,
        grid=grid,
        in_specs=[bn] * 8 + [bw((H, 3 * H)), bw((1, 3 * H))],
        out_specs=[bn] * 4,
        out_shape=[jax.ShapeDtypeStruct((N, H), jnp.float32)] * 4,
    )(x_agg, vec_dot, v3[0], v3[1], v3[2],
      vec_agg[:, 0, :], vec_agg[:, 1, :], vec_agg[:, 2, :],
      Wo, bo.reshape(1, 3 * H))


def kernel(x, vec, edge_index, r_ij, f_ij, d_ij, Wvec, Wq, bq, Wk, bk, Wv, bv,
           Wdk, bdk, Wdv, bdv, Ws, bs, Wf, bf, Wwsrc, Wwtrg, Wtsrc, Wttrg,
           Wo, bo):
    src, dst = edge_index[0], edge_index[1]
    (q, k, v, vec_dot, TT, A0, A1, A2, B0, B1, B2, v30, v31, v32) = \
        _node_precompute(x, vec, Wq, bq, Wk, bk, Wv, bv, Wvec, Wwtrg, Wwsrc,
                         Wttrg, Wtsrc)
    dk, dv, cut = _edge_dense(f_ij, r_ij, Wdk, bdk, Wdv, bdv)

    q_i = q[dst]
    k_j = k[src]
    v_j = v[src]
    TT_e = TT[dst]
    vj = vec[src]
    vi = vec[dst]
    Ai = jnp.stack([A0[dst], A1[dst], A2[dst]], axis=1)
    Bj = jnp.stack([B0[src], B1[src], B2[src]], axis=1)

    v_e, m0, m1, m2, df = _edge_mega(
        q_i, k_j, v_j, dk, dv, cut, f_ij, TT_e, vj, vi, Ai, Bj, d_ij,
        Ws, bs, Wf, bf, Wwtrg, Wwsrc, Wttrg, Wtsrc)

    x_agg = jax.ops.segment_sum(v_e, dst, num_segments=N)
    vec_agg = jax.ops.segment_sum(
        jnp.stack([m0, m1, m2], axis=1), dst, num_segments=N)

    dx, dv0, dv1, dv2 = _node_epilogue(x_agg, vec_dot, (v30, v31, v32),
                                       vec_agg, Wo, bo)
    dvec = jnp.stack([dv0, dv1, dv2], axis=1)
    return (dx, dvec, df)


# speedup vs baseline: 1.1752x; 1.1272x over previous
import functools
import jax
import jax.numpy as jnp
from jax import lax
from jax.experimental import pallas as pl
from jax.experimental.pallas import tpu as pltpu
from jax.experimental.pallas import tpu_sc as plsc

N = 10000
E = 160000
H = 128
HEADS = 8
HD = H // HEADS
CUTOFF = 5.0

NB = 1000
EB = 1600


def _silu(x):
    return x * jax.nn.sigmoid(x)


_GDN = lax.GatherDimensionNumbers(offset_dims=(), collapsed_slice_dims=(0,),
                                  start_index_map=(0,))


def _lane_perm(x, perm):
    return lax.gather(x, perm[:, None], dimension_numbers=_GDN,
                      slice_sizes=(1,),
                      mode=lax.GatherScatterMode.PROMISE_IN_BOUNDS)


def _lane_sum(x):
    lanes = lax.iota(jnp.int32, 16)
    for stq in (1, 2, 4, 8):
        x = x + _lane_perm(x, lanes ^ stq)
    return x


def _k1_body(x, v0, v1, v2, Wq, bq, Wk, bk, Wv, bv, Wvec, Wwtrg, Wwsrc,
             Wttrg, Wtsrc,
             qTT_o, kv_o, vdot_o, A_o, B_o, v3a_o, v3b_o, v3c_o):
    xb = x[...]
    qTT_o[:, :H] = xb @ Wq[...] + bq[...]
    kv_o[:, :H] = xb @ Wk[...] + bk[...]
    kv_o[:, H:] = xb @ Wv[...] + bv[...]
    vs = (v0[...], v1[...], v2[...])
    Wv3 = Wvec[...]
    vdot = jnp.zeros_like(xb)
    TT = jnp.zeros_like(xb)
    Wt1 = Wttrg[...]
    Wt2 = Wtsrc[...]
    Ww1 = Wwtrg[...]
    Ww2 = Wwsrc[...]
    v3_out = (v3a_o, v3b_o, v3c_o)
    for a in range(3):
        va = vs[a]
        v123 = va @ Wv3
        vdot = vdot + v123[:, :H] * v123[:, H:2 * H]
        v3_out[a][...] = v123[:, 2 * H:]
        TT = TT + (va @ Wt1) * (va @ Wt2)
        A_o[:, a * H:(a + 1) * H] = va @ Ww1
        B_o[:, a * H:(a + 1) * H] = va @ Ww2
    vdot_o[...] = vdot
    qTT_o[:, H:] = TT


def _node_precompute(x, vec, Wq, bq, Wk, bk, Wv, bv, Wvec, Wwtrg, Wwsrc,
                     Wttrg, Wtsrc):
    grid = (N // NB,)
    bn = pl.BlockSpec((NB, H), lambda i: (i, 0))
    b2 = pl.BlockSpec((NB, 2 * H), lambda i: (i, 0))
    b3 = pl.BlockSpec((NB, 3 * H), lambda i: (i, 0))
    bw = lambda s: pl.BlockSpec(s, lambda i: (0, 0))
    outs = [jax.ShapeDtypeStruct((N, 2 * H), jnp.float32),
            jax.ShapeDtypeStruct((N, 2 * H), jnp.float32),
            jax.ShapeDtypeStruct((N, H), jnp.float32),
            jax.ShapeDtypeStruct((N, 3 * H), jnp.float32),
            jax.ShapeDtypeStruct((N, 3 * H), jnp.float32),
            jax.ShapeDtypeStruct((N, H), jnp.float32),
            jax.ShapeDtypeStruct((N, H), jnp.float32),
            jax.ShapeDtypeStruct((N, H), jnp.float32)]
    return pl.pallas_call(
        _k1_body,
        grid=grid,
        in_specs=[bn, bn, bn, bn,
                  bw((H, H)), bw((1, H)), bw((H, H)), bw((1, H)),
                  bw((H, H)), bw((1, H)), bw((H, 3 * H)),
                  bw((H, H)), bw((H, H)), bw((H, H)), bw((H, H))],
        out_specs=[b2, b2, bn, b3, b3, bn, bn, bn],
        out_shape=outs,
    )(x, vec[:, 0, :], vec[:, 1, :], vec[:, 2, :],
      Wq, bq.reshape(1, H), Wk, bk.reshape(1, H), Wv, bv.reshape(1, H),
      Wvec, Wwtrg, Wwsrc, Wttrg, Wtsrc)


def _k2_body(f, r, Wdk, bdk, Wdv, bdv, dk_o, dv_o):
    fb = f[...]
    dk_o[...] = _silu(fb @ Wdk[...] + bdk[...])
    rb = r[...]
    cut = jnp.where(rb < CUTOFF,
                    0.5 * (jnp.cos(jnp.pi * rb / CUTOFF) + 1.0), 0.0)
    dv_o[...] = _silu(fb @ Wdv[...] + bdv[...]) * cut


def _edge_dense(f_ij, r_ij, Wdk, bdk, Wdv, bdv):
    grid = (E // EB,)
    be = pl.BlockSpec((EB, H), lambda i: (i, 0))
    b1 = pl.BlockSpec((EB, 1), lambda i: (i, 0))
    bw = lambda s: pl.BlockSpec(s, lambda i: (0, 0))
    return pl.pallas_call(
        _k2_body,
        grid=grid,
        in_specs=[be, b1, bw((H, H)), bw((1, H)), bw((H, H)), bw((1, H))],
        out_specs=[be, be],
        out_shape=[jax.ShapeDtypeStruct((E, H), jnp.float32),
                   jax.ShapeDtypeStruct((E, H), jnp.float32)],
    )(f_ij, r_ij.reshape(E, 1), Wdk, bdk.reshape(1, H), Wdv, bdv.reshape(1, H))


def _sc_edge_a(qTT, kv, dk, dv, src, dst):
    CH = 32
    NCHUNK = E // CH
    NW = 32
    base_chunks = NCHUNK // NW
    extra = NCHUNK - base_chunks * NW
    NPAD = 10240
    NT = NPAD // 16
    mesh = plsc.VectorSubcoreMesh(core_axis_name="c", subcore_axis_name="s")

    @functools.partial(
        pl.kernel, mesh=mesh,
        out_type=[jax.ShapeDtypeStruct((E, H), jnp.float32),
                  jax.ShapeDtypeStruct((E, H), jnp.float32),
                  jax.ShapeDtypeStruct((2 * N, H), jnp.float32)],
        scratch_types=[pltpu.VMEM((CH,), jnp.int32),
                       pltpu.VMEM((CH,), jnp.int32),
                       pltpu.VMEM((CH, 2 * H), jnp.float32),
                       pltpu.VMEM((CH, 2 * H), jnp.float32),
                       pltpu.VMEM((CH, H), jnp.float32),
                       pltpu.VMEM((CH, H), jnp.float32),
                       pltpu.VMEM((CH, H), jnp.float32),
                       pltpu.VMEM((CH, H), jnp.float32),
                       pltpu.VMEM_SHARED((NPAD, H), jnp.float32),
                       pltpu.SemaphoreType.DMA,
                       pltpu.SemaphoreType.DMA,
                       pltpu.SemaphoreType.DMA,
                       pltpu.SemaphoreType.DMA],
    )
    def k(qTT_h, kv_h, dk_h, dv_h, src_h, dst_h,
          ve_o, tte_o, xpart_o,
          dsti, srci, qb, kvb, dkb, dvb, veb, tteb, xsh,
          m1, m2, m3, m4):
        c = lax.axis_index("c")
        s = lax.axis_index("s")
        wid = c * 16 + s

        zv = jnp.zeros((16,), jnp.float32)

        def zr(i, _):
            for g in range(8):
                veb[i, pl.ds(g * 16, 16)] = zv
            return 0
        lax.fori_loop(0, CH, zr, 0)
        for z in range(NT // CH):
            pltpu.sync_copy(veb, xsh.at[pl.ds(s * NT + z * CH, CH)])
        plsc.subcore_barrier()

        nt = base_chunks + jnp.where(wid < extra, 1, 0)

        def chunk(ci, _):
            g = wid + ci * NW
            base = g * CH
            pltpu.sync_copy(dst_h.at[pl.ds(base, CH)], dsti)
            pltpu.sync_copy(src_h.at[pl.ds(base, CH)], srci)
            c1 = pltpu.async_copy(qTT_h.at[dsti], qb, m1)
            c2 = pltpu.async_copy(kv_h.at[srci], kvb, m2)
            c3 = pltpu.async_copy(dk_h.at[pl.ds(base, CH)], dkb, m3)
            c4 = pltpu.async_copy(dv_h.at[pl.ds(base, CH)], dvb, m4)
            c1.wait(); c2.wait(); c3.wait(); c4.wait()

            def edge(i, _):
                for h in range(8):
                    sl = pl.ds(16 * h, 16)
                    slv = pl.ds(H + 16 * h, 16)
                    t = qb[i, sl] * kvb[i, sl] * dkb[i, sl]
                    sb = _lane_sum(t)
                    av = sb / (1.0 + jnp.exp(-sb))
                    veb[i, sl] = kvb[i, slv] * dvb[i, sl] * av
                    tteb[i, sl] = qb[i, slv]
                return 0
            lax.fori_loop(0, CH, edge, 0)
            pltpu.sync_copy(veb, ve_o.at[pl.ds(base, CH)])
            pltpu.sync_copy(tteb, tte_o.at[pl.ds(base, CH)])
            pltpu.sync_copy(veb, xsh.at[dsti], add=True)
            return 0
        lax.fori_loop(0, nt, chunk, 0)
        plsc.subcore_barrier()
        last = N - 15 * NT

        @pl.when(s < 15)
        def _dump_full():
            pltpu.sync_copy(xsh.at[pl.ds(s * NT, NT)],
                            xpart_o.at[pl.ds(c * N + s * NT, NT)])

        @pl.when(s == 15)
        def _dump_last():
            pltpu.sync_copy(xsh.at[pl.ds(15 * NT, last)],
                            xpart_o.at[pl.ds(c * N + 15 * NT, last)])

    return k(qTT, kv, dk, dv, src, dst)


def _k3_body(ve, Ws, bs, s_o):
    s_o[...] = _silu(ve[...] @ Ws[...] + bs[...])


def _s_kernel(v_e, Ws, bs):
    grid = (E // EB,)
    be = pl.BlockSpec((EB, H), lambda i: (i, 0))
    b2 = pl.BlockSpec((EB, 2 * H), lambda i: (i, 0))
    bw = lambda s: pl.BlockSpec(s, lambda i: (0, 0))
    return pl.pallas_call(
        _k3_body,
        grid=grid,
        in_specs=[be, bw((H, 2 * H)), bw((1, 2 * H))],
        out_specs=b2,
        out_shape=jax.ShapeDtypeStruct((E, 2 * H), jnp.float32),
    )(v_e, Ws, bs.reshape(1, 2 * H))


def _kb_body(vj, vi, Ai, Bj, sref, d0, d1, d2,
             m_o, wt1_o, dvi_o, dvj_o):
    s1 = sref[:, :H]
    s2 = sref[:, H:]
    ds = (d0[...], d1[...], d2[...])
    wt1 = jnp.zeros_like(s1)
    dvi = jnp.zeros_like(s1)
    dvj = jnp.zeros_like(s1)
    for a in range(3):
        sl = slice(a * H, (a + 1) * H)
        vja = vj[:, sl]
        via = vi[:, sl]
        m_o[:, sl] = vja * s1 + s2 * ds[a]
        dvi = dvi + ds[a] * via
        dvj = dvj + ds[a] * vja
        wt1 = wt1 + Ai[:, sl] * Bj[:, sl]
    wt1_o[...] = wt1
    dvi_o[...] = dvi
    dvj_o[...] = dvj


def _stage_b_tc(vj, vi, Ai, Bj, s, d_ij):
    grid = (E // EB,)
    be = pl.BlockSpec((EB, H), lambda i: (i, 0))
    b1 = pl.BlockSpec((EB, 1), lambda i: (i, 0))
    b2 = pl.BlockSpec((EB, 2 * H), lambda i: (i, 0))
    b3 = pl.BlockSpec((EB, 3 * H), lambda i: (i, 0))
    return pl.pallas_call(
        _kb_body,
        grid=grid,
        in_specs=[b3, b3, b3, b3, b2, b1, b1, b1],
        out_specs=[b3, be, be, be],
        out_shape=[jax.ShapeDtypeStruct((E, 3 * H), jnp.float32),
                   jax.ShapeDtypeStruct((E, H), jnp.float32),
                   jax.ShapeDtypeStruct((E, H), jnp.float32),
                   jax.ShapeDtypeStruct((E, H), jnp.float32)],
    )(vj, vi, Ai, Bj, s, d_ij[:, 0:1], d_ij[:, 1:2], d_ij[:, 2:3])


def _k4_body(f, wt1, dvi, dvj, tte, Wf, bf, Wwtrg, Wwsrc, Wttrg, Wtsrc,
             df_o):
    dvi_b = dvi[...]
    w_dot = wt1[...] - (dvi_b @ Wwtrg[...]) * (dvj[...] @ Wwsrc[...])
    t_dot = tte[...] - (dvi_b @ Wttrg[...]) * (dvi_b @ Wtsrc[...])
    ff = _silu(f[...] @ Wf[...] + bf[...])
    df_o[...] = ff[:, :H] * w_dot + ff[:, H:] * t_dot


def _df_kernel(f_ij, wt1, dvi, dvj, tte, Wf, bf, Wwtrg, Wwsrc, Wttrg, Wtsrc):
    grid = (E // EB,)
    be = pl.BlockSpec((EB, H), lambda i: (i, 0))
    bw = lambda s: pl.BlockSpec(s, lambda i: (0, 0))
    return pl.pallas_call(
        _k4_body,
        grid=grid,
        in_specs=[be, be, be, be, be,
                  bw((H, 2 * H)), bw((1, 2 * H)),
                  bw((H, H)), bw((H, H)), bw((H, H)), bw((H, H))],
        out_specs=be,
        out_shape=jax.ShapeDtypeStruct((E, H), jnp.float32),
    )(f_ij, wt1, dvi, dvj, tte, Wf, bf.reshape(1, 2 * H),
      Wwtrg, Wwsrc, Wttrg, Wtsrc)


def _k5_body(xp0, xp1, vdot, v30, v31, v32, g0, g1, g2, Wo, bo,
             dx_o, dv0_o, dv1_o, dv2_o):
    o = (xp0[...] + xp1[...]) @ Wo[...] + bo[...]
    o1, o2, o3 = o[:, :H], o[:, H:2 * H], o[:, 2 * H:]
    dx_o[...] = vdot[...] * o2 + o3
    dv0_o[...] = v30[...] * o1 + g0[...]
    dv1_o[...] = v31[...] * o1 + g1[...]
    dv2_o[...] = v32[...] * o1 + g2[...]


def _node_epilogue(xpart, vec_dot, v3, vec_agg, Wo, bo):
    grid = (N // NB,)
    bn = pl.BlockSpec((NB, H), lambda i: (i, 0))
    bnR = pl.BlockSpec((NB, H), lambda i: (N // NB + i, 0))
    bw = lambda s: pl.BlockSpec(s, lambda i: (0, 0))
    return pl.pallas_call(
        _k5_body,
        grid=grid,
        in_specs=[bn, bnR, bn, bn, bn, bn, bn, bn, bn,
                  bw((H, 3 * H)), bw((1, 3 * H))],
        out_specs=[bn] * 4,
        out_shape=[jax.ShapeDtypeStruct((N, H), jnp.float32)] * 4,
    )(xpart, xpart, vec_dot, v3[0], v3[1], v3[2],
      vec_agg[:, 0, :], vec_agg[:, 1, :], vec_agg[:, 2, :],
      Wo, bo.reshape(1, 3 * H))


def kernel(x, vec, edge_index, r_ij, f_ij, d_ij, Wvec, Wq, bq, Wk, bk, Wv, bv,
           Wdk, bdk, Wdv, bdv, Ws, bs, Wf, bf, Wwsrc, Wwtrg, Wtsrc, Wttrg,
           Wo, bo):
    src, dst = edge_index[0], edge_index[1]
    (qTT, kv, vec_dot, A_cat, B_cat, v30, v31, v32) = \
        _node_precompute(x, vec, Wq, bq, Wk, bk, Wv, bv, Wvec, Wwtrg, Wwsrc,
                         Wttrg, Wtsrc)
    dk, dv = _edge_dense(f_ij, r_ij, Wdk, bdk, Wdv, bdv)

    v_e, TT_e, xpart = _sc_edge_a(qTT, kv, dk, dv, src, dst)

    s = _s_kernel(v_e, Ws, bs)

    vec_cat = vec.reshape(N, 3 * H)
    vj = vec_cat[src]
    vi = vec_cat[dst]
    Ai = A_cat[dst]
    Bj = B_cat[src]
    msg, wt1, dvi, dvj = _stage_b_tc(vj, vi, Ai, Bj, s, d_ij)
    vec_agg = jax.ops.segment_sum(
        msg.reshape(E, 3, H), dst, num_segments=N)

    df = _df_kernel(f_ij, wt1, dvi, dvj, TT_e, Wf, bf, Wwtrg, Wwsrc,
                    Wttrg, Wtsrc)

    dx, dv0, dv1, dv2 = _node_epilogue(xpart, vec_dot, (v30, v31, v32),
                                       vec_agg, Wo, bo)
    dvec = jnp.stack([dv0, dv1, dv2], axis=1)
    return (dx, dvec, df)


# speedup vs baseline: 4.9373x; 4.2012x over previous
import functools
import jax
import jax.numpy as jnp
from jax import lax
from jax.experimental import pallas as pl
from jax.experimental.pallas import tpu as pltpu
from jax.experimental.pallas import tpu_sc as plsc

N = 10000
E = 160000
H = 128
HEADS = 8
HD = H // HEADS
CUTOFF = 5.0

NB = 1000
EB = 1600


def _silu(x):
    return x * jax.nn.sigmoid(x)


_GDN = lax.GatherDimensionNumbers(offset_dims=(), collapsed_slice_dims=(0,),
                                  start_index_map=(0,))


def _lane_perm(x, perm):
    return lax.gather(x, perm[:, None], dimension_numbers=_GDN,
                      slice_sizes=(1,),
                      mode=lax.GatherScatterMode.PROMISE_IN_BOUNDS)


def _lane_sum(x):
    lanes = lax.iota(jnp.int32, 16)
    for stq in (1, 2, 4, 8):
        x = x + _lane_perm(x, lanes ^ stq)
    return x


def _k1_body(x, v0, v1, v2, Wq, bq, Wk, bk, Wv, bv, Wvec, Wwtrg, Wwsrc,
             Wttrg, Wtsrc,
             qTT_o, kv_o, vdot_o, A_o, B_o, v3a_o, v3b_o, v3c_o):
    xb = x[...]
    qTT_o[:, :H] = xb @ Wq[...] + bq[...]
    kv_o[:, :H] = xb @ Wk[...] + bk[...]
    kv_o[:, H:] = xb @ Wv[...] + bv[...]
    vs = (v0[...], v1[...], v2[...])
    Wv3 = Wvec[...]
    vdot = jnp.zeros_like(xb)
    TT = jnp.zeros_like(xb)
    Wt1 = Wttrg[...]
    Wt2 = Wtsrc[...]
    Ww1 = Wwtrg[...]
    Ww2 = Wwsrc[...]
    v3_out = (v3a_o, v3b_o, v3c_o)
    for a in range(3):
        va = vs[a]
        v123 = va @ Wv3
        vdot = vdot + v123[:, :H] * v123[:, H:2 * H]
        v3_out[a][...] = v123[:, 2 * H:]
        TT = TT + (va @ Wt1) * (va @ Wt2)
        A_o[:, a * H:(a + 1) * H] = va @ Ww1
        B_o[:, a * H:(a + 1) * H] = va @ Ww2
    vdot_o[...] = vdot
    qTT_o[:, H:] = TT


def _node_precompute(x, vec, Wq, bq, Wk, bk, Wv, bv, Wvec, Wwtrg, Wwsrc,
                     Wttrg, Wtsrc):
    grid = (N // NB,)
    bn = pl.BlockSpec((NB, H), lambda i: (i, 0))
    b2 = pl.BlockSpec((NB, 2 * H), lambda i: (i, 0))
    b3 = pl.BlockSpec((NB, 3 * H), lambda i: (i, 0))
    bw = lambda s: pl.BlockSpec(s, lambda i: (0, 0))
    outs = [jax.ShapeDtypeStruct((N, 2 * H), jnp.float32),
            jax.ShapeDtypeStruct((N, 2 * H), jnp.float32),
            jax.ShapeDtypeStruct((N, H), jnp.float32),
            jax.ShapeDtypeStruct((N, 3 * H), jnp.float32),
            jax.ShapeDtypeStruct((N, 3 * H), jnp.float32),
            jax.ShapeDtypeStruct((N, H), jnp.float32),
            jax.ShapeDtypeStruct((N, H), jnp.float32),
            jax.ShapeDtypeStruct((N, H), jnp.float32)]
    return pl.pallas_call(
        _k1_body,
        grid=grid,
        in_specs=[bn, bn, bn, bn,
                  bw((H, H)), bw((1, H)), bw((H, H)), bw((1, H)),
                  bw((H, H)), bw((1, H)), bw((H, 3 * H)),
                  bw((H, H)), bw((H, H)), bw((H, H)), bw((H, H))],
        out_specs=[b2, b2, bn, b3, b3, bn, bn, bn],
        out_shape=outs,
    )(x, vec[:, 0, :], vec[:, 1, :], vec[:, 2, :],
      Wq, bq.reshape(1, H), Wk, bk.reshape(1, H), Wv, bv.reshape(1, H),
      Wvec, Wwtrg, Wwsrc, Wttrg, Wtsrc)


def _k2_body(f, r, Wdk, bdk, Wdv, bdv, dk_o, dv_o):
    fb = f[...]
    dk_o[...] = _silu(fb @ Wdk[...] + bdk[...])
    rb = r[...]
    cut = jnp.where(rb < CUTOFF,
                    0.5 * (jnp.cos(jnp.pi * rb / CUTOFF) + 1.0), 0.0)
    dv_o[...] = _silu(fb @ Wdv[...] + bdv[...]) * cut


def _edge_dense(f_ij, r_ij, Wdk, bdk, Wdv, bdv):
    grid = (E // EB,)
    be = pl.BlockSpec((EB, H), lambda i: (i, 0))
    b1 = pl.BlockSpec((EB, 1), lambda i: (i, 0))
    bw = lambda s: pl.BlockSpec(s, lambda i: (0, 0))
    return pl.pallas_call(
        _k2_body,
        grid=grid,
        in_specs=[be, b1, bw((H, H)), bw((1, H)), bw((H, H)), bw((1, H))],
        out_specs=[be, be],
        out_shape=[jax.ShapeDtypeStruct((E, H), jnp.float32),
                   jax.ShapeDtypeStruct((E, H), jnp.float32)],
    )(f_ij, r_ij.reshape(E, 1), Wdk, bdk.reshape(1, H), Wdv, bdv.reshape(1, H))


def _sc_edge_a(qTT, kv, dk, dv, src, dst):
    CH = 32
    NCHUNK = E // CH
    NW = 32
    base_chunks = NCHUNK // NW
    extra = NCHUNK - base_chunks * NW
    NPAD = 10240
    NT = NPAD // 16
    mesh = plsc.VectorSubcoreMesh(core_axis_name="c", subcore_axis_name="s")

    @functools.partial(
        pl.kernel, mesh=mesh,
        out_type=[jax.ShapeDtypeStruct((E, H), jnp.float32),
                  jax.ShapeDtypeStruct((E, H), jnp.float32),
                  jax.ShapeDtypeStruct((2 * N, H), jnp.float32)],
        scratch_types=[pltpu.VMEM((CH,), jnp.int32),
                       pltpu.VMEM((CH,), jnp.int32),
                       pltpu.VMEM((CH, 2 * H), jnp.float32),
                       pltpu.VMEM((CH, 2 * H), jnp.float32),
                       pltpu.VMEM((CH, H), jnp.float32),
                       pltpu.VMEM((CH, H), jnp.float32),
                       pltpu.VMEM((CH, H), jnp.float32),
                       pltpu.VMEM((CH, H), jnp.float32),
                       pltpu.VMEM_SHARED((NPAD, H), jnp.float32),
                       pltpu.SemaphoreType.DMA,
                       pltpu.SemaphoreType.DMA,
                       pltpu.SemaphoreType.DMA,
                       pltpu.SemaphoreType.DMA],
    )
    def k(qTT_h, kv_h, dk_h, dv_h, src_h, dst_h,
          ve_o, tte_o, xpart_o,
          dsti, srci, qb, kvb, dkb, dvb, veb, tteb, xsh,
          m1, m2, m3, m4):
        c = lax.axis_index("c")
        s = lax.axis_index("s")
        wid = c * 16 + s

        zv = jnp.zeros((16,), jnp.float32)

        def zr(i, _):
            for g in range(8):
                veb[i, pl.ds(g * 16, 16)] = zv
            return 0
        lax.fori_loop(0, CH, zr, 0)
        for z in range(NT // CH):
            pltpu.sync_copy(veb, xsh.at[pl.ds(s * NT + z * CH, CH)])
        plsc.subcore_barrier()

        nt = base_chunks + jnp.where(wid < extra, 1, 0)

        def chunk(ci, _):
            g = wid + ci * NW
            base = g * CH
            pltpu.sync_copy(dst_h.at[pl.ds(base, CH)], dsti)
            pltpu.sync_copy(src_h.at[pl.ds(base, CH)], srci)
            c1 = pltpu.async_copy(qTT_h.at[dsti], qb, m1)
            c2 = pltpu.async_copy(kv_h.at[srci], kvb, m2)
            c3 = pltpu.async_copy(dk_h.at[pl.ds(base, CH)], dkb, m3)
            c4 = pltpu.async_copy(dv_h.at[pl.ds(base, CH)], dvb, m4)
            c1.wait(); c2.wait(); c3.wait(); c4.wait()

            def edge(i, _):
                for h in range(8):
                    sl = pl.ds(16 * h, 16)
                    slv = pl.ds(H + 16 * h, 16)
                    t = qb[i, sl] * kvb[i, sl] * dkb[i, sl]
                    sb = _lane_sum(t)
                    av = sb / (1.0 + jnp.exp(-sb))
                    veb[i, sl] = kvb[i, slv] * dvb[i, sl] * av
                    tteb[i, sl] = qb[i, slv]
                return 0
            lax.fori_loop(0, CH, edge, 0)
            pltpu.sync_copy(veb, ve_o.at[pl.ds(base, CH)])
            pltpu.sync_copy(tteb, tte_o.at[pl.ds(base, CH)])
            pltpu.sync_copy(veb, xsh.at[dsti], add=True)
            return 0
        lax.fori_loop(0, nt, chunk, 0)
        plsc.subcore_barrier()
        last = N - 15 * NT

        @pl.when(s < 15)
        def _dump_full():
            pltpu.sync_copy(xsh.at[pl.ds(s * NT, NT)],
                            xpart_o.at[pl.ds(c * N + s * NT, NT)])

        @pl.when(s == 15)
        def _dump_last():
            pltpu.sync_copy(xsh.at[pl.ds(15 * NT, last)],
                            xpart_o.at[pl.ds(c * N + 15 * NT, last)])

    return k(qTT, kv, dk, dv, src, dst)


def _k3_body(ve, d0, d1, d2, Ws, bs, s1_o, sd0_o, sd1_o, sd2_o, d48_o):
    s = _silu(ve[...] @ Ws[...] + bs[...])
    s1_o[...] = s[:, :H]
    s2 = s[:, H:]
    sd_out = (sd0_o, sd1_o, sd2_o)
    ds = (d0[...], d1[...], d2[...])
    for a in range(3):
        sd_out[a][...] = s2 * ds[a]
        d48_o[:, a * 16:(a + 1) * 16] = jnp.broadcast_to(ds[a], (EB, 16))


def _s_kernel(v_e, d_ij, Ws, bs):
    grid = (E // EB,)
    be = pl.BlockSpec((EB, H), lambda i: (i, 0))
    b1 = pl.BlockSpec((EB, 1), lambda i: (i, 0))
    bd = pl.BlockSpec((EB, 48), lambda i: (i, 0))
    bw = lambda s: pl.BlockSpec(s, lambda i: (0, 0))
    return pl.pallas_call(
        _k3_body,
        grid=grid,
        in_specs=[be, b1, b1, b1, bw((H, 2 * H)), bw((1, 2 * H))],
        out_specs=[be, be, be, be, bd],
        out_shape=[jax.ShapeDtypeStruct((E, H), jnp.float32)] * 4 +
                  [jax.ShapeDtypeStruct((E, 48), jnp.float32)],
    )(v_e, d_ij[:, 0:1], d_ij[:, 1:2], d_ij[:, 2:3], Ws,
      bs.reshape(1, 2 * H))


def _sc_msg_scatter(vec0, vec1, vec2, s1, sd0, sd1, sd2, src, dst):
    CH1 = 64
    NCHUNK = E // CH1
    NW = 32
    base_chunks = NCHUNK // NW
    extra = NCHUNK - base_chunks * NW
    NPAD = 10240
    NT = NPAD // 16
    mesh = plsc.VectorSubcoreMesh(core_axis_name="c", subcore_axis_name="s")

    @functools.partial(
        pl.kernel, mesh=mesh,
        out_type=jax.ShapeDtypeStruct((6 * N, H), jnp.float32),
        scratch_types=[pltpu.VMEM((CH1,), jnp.int32),
                       pltpu.VMEM((CH1,), jnp.int32),
                       pltpu.VMEM((CH1, H), jnp.float32),
                       pltpu.VMEM((CH1, H), jnp.float32),
                       pltpu.VMEM((CH1, H), jnp.float32),
                       pltpu.VMEM((CH1, H), jnp.float32),
                       pltpu.VMEM_SHARED((NPAD, H), jnp.float32),
                       pltpu.SemaphoreType.DMA,
                       pltpu.SemaphoreType.DMA,
                       pltpu.SemaphoreType.DMA],
    )
    def k(v0_h, v1_h, v2_h, s1_h, sd0_h, sd1_h, sd2_h, src_h, dst_h,
          msum_o, dsti, srci, vecb, s1b, sdb, msgb, vsh, m1, m2, m3):
        c = lax.axis_index("c")
        s = lax.axis_index("s")
        wid = c * 16 + s
        vtabs = (v0_h, v1_h, v2_h)
        sdtabs = (sd0_h, sd1_h, sd2_h)
        zv = jnp.zeros((16,), jnp.float32)
        last = N - 15 * NT

        def zbuf(i, _):
            for g in range(8):
                msgb[i, pl.ds(g * 16, 16)] = zv
            return 0

        def zero_spmem():
            lax.fori_loop(0, CH1, zbuf, 0)
            for z in range(NT // CH1):
                pltpu.sync_copy(msgb, vsh.at[pl.ds(s * NT + z * CH1, CH1)])

        zero_spmem()
        plsc.subcore_barrier()

        nt = base_chunks + jnp.where(wid < extra, 1, 0)
        for a in range(3):
            vtab = vtabs[a]
            sdtab = sdtabs[a]

            def chunk(ci, _):
                g = wid + ci * NW
                base = g * CH1
                pltpu.sync_copy(dst_h.at[pl.ds(base, CH1)], dsti)
                pltpu.sync_copy(src_h.at[pl.ds(base, CH1)], srci)
                c1 = pltpu.async_copy(vtab.at[srci], vecb, m1)
                c2 = pltpu.async_copy(s1_h.at[pl.ds(base, CH1)], s1b, m2)
                c3 = pltpu.async_copy(sdtab.at[pl.ds(base, CH1)], sdb, m3)
                c1.wait(); c2.wait(); c3.wait()

                def edge(i, _):
                    for g8 in range(8):
                        sl = pl.ds(g8 * 16, 16)
                        msgb[i, sl] = vecb[i, sl] * s1b[i, sl] + sdb[i, sl]
                    return 0
                lax.fori_loop(0, CH1, edge, 0)
                pltpu.sync_copy(msgb, vsh.at[dsti], add=True)
                return 0
            lax.fori_loop(0, nt, chunk, 0)
            plsc.subcore_barrier()
            row0 = (c * 3 + a) * N

            @pl.when(s < 15)
            def _dump_full():
                pltpu.sync_copy(vsh.at[pl.ds(s * NT, NT)],
                                msum_o.at[pl.ds(row0 + s * NT, NT)])

            @pl.when(s == 15)
            def _dump_last():
                pltpu.sync_copy(vsh.at[pl.ds(15 * NT, last)],
                                msum_o.at[pl.ds(row0 + 15 * NT, last)])
            if a < 2:
                zero_spmem()
            plsc.subcore_barrier()

    return k(vec0, vec1, vec2, s1, sd0, sd1, sd2, src, dst)


def _sc_wdot(TabD, TabS, d48, src, dst):
    CH2 = 40
    NCHUNK = E // CH2
    NW = 32
    per_w = NCHUNK // NW
    mesh = plsc.VectorSubcoreMesh(core_axis_name="c", subcore_axis_name="s")

    @functools.partial(
        pl.kernel, mesh=mesh,
        out_type=[jax.ShapeDtypeStruct((E, H), jnp.float32)] * 3,
        scratch_types=[pltpu.VMEM((CH2,), jnp.int32),
                       pltpu.VMEM((CH2,), jnp.int32),
                       pltpu.VMEM((CH2, 6 * H), jnp.float32),
                       pltpu.VMEM((CH2, 6 * H), jnp.float32),
                       pltpu.VMEM((CH2, 48), jnp.float32),
                       pltpu.VMEM((CH2, H), jnp.float32),
                       pltpu.VMEM((CH2, H), jnp.float32),
                       pltpu.VMEM((CH2, H), jnp.float32),
                       pltpu.SemaphoreType.DMA,
                       pltpu.SemaphoreType.DMA,
                       pltpu.SemaphoreType.DMA],
    )
    def k(TabD_h, TabS_h, d48_h, src_h, dst_h,
          wt1_o, dvi_o, dvj_o,
          dsti, srci, Db, Sb, d48b, wt1b, dvib, dvjb, m1, m2, m3):
        c = lax.axis_index("c")
        s = lax.axis_index("s")
        wid = c * 16 + s

        def chunk(ci, _):
            g = wid + ci * NW
            base = g * CH2
            pltpu.sync_copy(dst_h.at[pl.ds(base, CH2)], dsti)
            pltpu.sync_copy(src_h.at[pl.ds(base, CH2)], srci)
            c1 = pltpu.async_copy(TabD_h.at[dsti], Db, m1)
            c2 = pltpu.async_copy(TabS_h.at[srci], Sb, m2)
            c3 = pltpu.async_copy(d48_h.at[pl.ds(base, CH2)], d48b, m3)
            c1.wait(); c2.wait(); c3.wait()

            def edge(i, _):
                d0v = d48b[i, pl.ds(0, 16)]
                d1v = d48b[i, pl.ds(16, 16)]
                d2v = d48b[i, pl.ds(32, 16)]
                for g8 in range(8):
                    o = g8 * 16
                    sl = pl.ds(o, 16)
                    vi0 = Db[i, pl.ds(o, 16)]
                    vi1 = Db[i, pl.ds(H + o, 16)]
                    vi2 = Db[i, pl.ds(2 * H + o, 16)]
                    A0 = Db[i, pl.ds(3 * H + o, 16)]
                    A1 = Db[i, pl.ds(4 * H + o, 16)]
                    A2 = Db[i, pl.ds(5 * H + o, 16)]
                    vj0 = Sb[i, pl.ds(o, 16)]
                    vj1 = Sb[i, pl.ds(H + o, 16)]
                    vj2 = Sb[i, pl.ds(2 * H + o, 16)]
                    B0 = Sb[i, pl.ds(3 * H + o, 16)]
                    B1 = Sb[i, pl.ds(4 * H + o, 16)]
                    B2 = Sb[i, pl.ds(5 * H + o, 16)]
                    wt1b[i, sl] = A0 * B0 + A1 * B1 + A2 * B2
                    dvib[i, sl] = d0v * vi0 + d1v * vi1 + d2v * vi2
                    dvjb[i, sl] = d0v * vj0 + d1v * vj1 + d2v * vj2
                return 0
            lax.fori_loop(0, CH2, edge, 0)
            pltpu.sync_copy(wt1b, wt1_o.at[pl.ds(base, CH2)])
            pltpu.sync_copy(dvib, dvi_o.at[pl.ds(base, CH2)])
            pltpu.sync_copy(dvjb, dvj_o.at[pl.ds(base, CH2)])
            return 0
        lax.fori_loop(0, per_w, chunk, 0)

    return k(TabD, TabS, d48, src, dst)


def _k4_body(f, wt1, dvi, dvj, tte, Wf, bf, Wwtrg, Wwsrc, Wttrg, Wtsrc,
             df_o):
    dvi_b = dvi[...]
    w_dot = wt1[...] - (dvi_b @ Wwtrg[...]) * (dvj[...] @ Wwsrc[...])
    t_dot = tte[...] - (dvi_b @ Wttrg[...]) * (dvi_b @ Wtsrc[...])
    ff = _silu(f[...] @ Wf[...] + bf[...])
    df_o[...] = ff[:, :H] * w_dot + ff[:, H:] * t_dot


def _df_kernel(f_ij, wt1, dvi, dvj, tte, Wf, bf, Wwtrg, Wwsrc, Wttrg, Wtsrc):
    grid = (E // EB,)
    be = pl.BlockSpec((EB, H), lambda i: (i, 0))
    bw = lambda s: pl.BlockSpec(s, lambda i: (0, 0))
    return pl.pallas_call(
        _k4_body,
        grid=grid,
        in_specs=[be, be, be, be, be,
                  bw((H, 2 * H)), bw((1, 2 * H)),
                  bw((H, H)), bw((H, H)), bw((H, H)), bw((H, H))],
        out_specs=be,
        out_shape=jax.ShapeDtypeStruct((E, H), jnp.float32),
    )(f_ij, wt1, dvi, dvj, tte, Wf, bf.reshape(1, 2 * H),
      Wwtrg, Wwsrc, Wttrg, Wtsrc)


def _k5_body(xp0, xp1, vdot, v30, v31, v32, ga0, gb0, ga1, gb1, ga2, gb2,
             Wo, bo, dx_o, dv0_o, dv1_o, dv2_o):
    o = (xp0[...] + xp1[...]) @ Wo[...] + bo[...]
    o1, o2, o3 = o[:, :H], o[:, H:2 * H], o[:, 2 * H:]
    dx_o[...] = vdot[...] * o2 + o3
    dv0_o[...] = v30[...] * o1 + ga0[...] + gb0[...]
    dv1_o[...] = v31[...] * o1 + ga1[...] + gb1[...]
    dv2_o[...] = v32[...] * o1 + ga2[...] + gb2[...]


def _node_epilogue(xpart, vec_dot, v3, msum, Wo, bo):
    grid = (N // NB,)
    nb = N // NB
    bn = pl.BlockSpec((NB, H), lambda i: (i, 0))
    bnR = pl.BlockSpec((NB, H), lambda i: (nb + i, 0))

    def bm(a, c):
        return pl.BlockSpec((NB, H),
                            lambda i, _a=a, _c=c: ((_c * 3 + _a) * nb + i, 0))
    bw = lambda s: pl.BlockSpec(s, lambda i: (0, 0))
    return pl.pallas_call(
        _k5_body,
        grid=grid,
        in_specs=[bn, bnR, bn, bn, bn, bn,
                  bm(0, 0), bm(0, 1), bm(1, 0), bm(1, 1), bm(2, 0), bm(2, 1),
                  bw((H, 3 * H)), bw((1, 3 * H))],
        out_specs=[bn] * 4,
        out_shape=[jax.ShapeDtypeStruct((N, H), jnp.float32)] * 4,
    )(xpart, xpart, vec_dot, v3[0], v3[1], v3[2],
      msum, msum, msum, msum, msum, msum,
      Wo, bo.reshape(1, 3 * H))


def kernel(x, vec, edge_index, r_ij, f_ij, d_ij, Wvec, Wq, bq, Wk, bk, Wv, bv,
           Wdk, bdk, Wdv, bdv, Ws, bs, Wf, bf, Wwsrc, Wwtrg, Wtsrc, Wttrg,
           Wo, bo):
    src, dst = edge_index[0], edge_index[1]
    (qTT, kv, vec_dot, A_cat, B_cat, v30, v31, v32) = \
        _node_precompute(x, vec, Wq, bq, Wk, bk, Wv, bv, Wvec, Wwtrg, Wwsrc,
                         Wttrg, Wtsrc)
    dk, dv = _edge_dense(f_ij, r_ij, Wdk, bdk, Wdv, bdv)

    v_e, TT_e, xpart = _sc_edge_a(qTT, kv, dk, dv, src, dst)

    s1, sd0, sd1, sd2, d48 = _s_kernel(v_e, d_ij, Ws, bs)

    vec_cat = vec.reshape(N, 3 * H)
    TabD = jnp.concatenate([vec_cat, A_cat], axis=1)
    TabS = jnp.concatenate([vec_cat, B_cat], axis=1)
    msum = _sc_msg_scatter(vec[:, 0, :], vec[:, 1, :], vec[:, 2, :],
                           s1, sd0, sd1, sd2, src, dst)
    wt1, dvi, dvj = _sc_wdot(TabD, TabS, d48, src, dst)

    df = _df_kernel(f_ij, wt1, dvi, dvj, TT_e, Wf, bf, Wwtrg, Wwsrc,
                    Wttrg, Wtsrc)

    dx, dv0, dv1, dv2 = _node_epilogue(xpart, vec_dot, (v30, v31, v32),
                                       msum, Wo, bo)
    dvec = jnp.stack([dv0, dv1, dv2], axis=1)
    return (dx, dvec, df)


# speedup vs baseline: 5.3826x; 1.0902x over previous
import functools
import jax
import jax.numpy as jnp
from jax import lax
from jax.experimental import pallas as pl
from jax.experimental.pallas import tpu as pltpu
from jax.experimental.pallas import tpu_sc as plsc

N = 10000
E = 160000
H = 128
HEADS = 8
HD = H // HEADS
CUTOFF = 5.0

NB = 1000
EB = 1600


def _silu(x):
    return x * jax.nn.sigmoid(x)


_GDN = lax.GatherDimensionNumbers(offset_dims=(), collapsed_slice_dims=(0,),
                                  start_index_map=(0,))


def _lane_perm(x, perm):
    return lax.gather(x, perm[:, None], dimension_numbers=_GDN,
                      slice_sizes=(1,),
                      mode=lax.GatherScatterMode.PROMISE_IN_BOUNDS)


def _lane_sum(x):
    lanes = lax.iota(jnp.int32, 16)
    for stq in (1, 2, 4, 8):
        x = x + _lane_perm(x, lanes ^ stq)
    return x


def _k1_body(x, v0, v1, v2, Wq, bq, Wk, bk, Wv, bv, Wvec, Wwtrg, Wwsrc,
             Wttrg, Wtsrc,
             qTT_o, kv_o, vdot_o, A_o, B_o, v3a_o, v3b_o, v3c_o):
    xb = x[...]
    qTT_o[:, :H] = xb @ Wq[...] + bq[...]
    kv_o[:, :H] = xb @ Wk[...] + bk[...]
    kv_o[:, H:] = xb @ Wv[...] + bv[...]
    vs = (v0[...], v1[...], v2[...])
    Wv3 = Wvec[...]
    vdot = jnp.zeros_like(xb)
    TT = jnp.zeros_like(xb)
    Wt1 = Wttrg[...]
    Wt2 = Wtsrc[...]
    Ww1 = Wwtrg[...]
    Ww2 = Wwsrc[...]
    v3_out = (v3a_o, v3b_o, v3c_o)
    for a in range(3):
        va = vs[a]
        v123 = va @ Wv3
        vdot = vdot + v123[:, :H] * v123[:, H:2 * H]
        v3_out[a][...] = v123[:, 2 * H:]
        TT = TT + (va @ Wt1) * (va @ Wt2)
        A_o[:, a * H:(a + 1) * H] = va @ Ww1
        B_o[:, a * H:(a + 1) * H] = va @ Ww2
    vdot_o[...] = vdot
    qTT_o[:, H:] = TT


def _node_precompute(x, vec, Wq, bq, Wk, bk, Wv, bv, Wvec, Wwtrg, Wwsrc,
                     Wttrg, Wtsrc):
    grid = (N // NB,)
    bn = pl.BlockSpec((NB, H), lambda i: (i, 0))
    b2 = pl.BlockSpec((NB, 2 * H), lambda i: (i, 0))
    b3 = pl.BlockSpec((NB, 3 * H), lambda i: (i, 0))
    bw = lambda s: pl.BlockSpec(s, lambda i: (0, 0))
    outs = [jax.ShapeDtypeStruct((N, 2 * H), jnp.float32),
            jax.ShapeDtypeStruct((N, 2 * H), jnp.float32),
            jax.ShapeDtypeStruct((N, H), jnp.float32),
            jax.ShapeDtypeStruct((N, 3 * H), jnp.float32),
            jax.ShapeDtypeStruct((N, 3 * H), jnp.float32),
            jax.ShapeDtypeStruct((N, H), jnp.float32),
            jax.ShapeDtypeStruct((N, H), jnp.float32),
            jax.ShapeDtypeStruct((N, H), jnp.float32)]
    return pl.pallas_call(
        _k1_body,
        grid=grid,
        in_specs=[bn, bn, bn, bn,
                  bw((H, H)), bw((1, H)), bw((H, H)), bw((1, H)),
                  bw((H, H)), bw((1, H)), bw((H, 3 * H)),
                  bw((H, H)), bw((H, H)), bw((H, H)), bw((H, H))],
        out_specs=[b2, b2, bn, b3, b3, bn, bn, bn],
        out_shape=outs,
    )(x, vec[:, 0, :], vec[:, 1, :], vec[:, 2, :],
      Wq, bq.reshape(1, H), Wk, bk.reshape(1, H), Wv, bv.reshape(1, H),
      Wvec, Wwtrg, Wwsrc, Wttrg, Wtsrc)


def _k2_body(f, r, Wdk, bdk, Wdv, bdv, dkv_o):
    fb = f[...]
    dkv_o[:, :H] = _silu(fb @ Wdk[...] + bdk[...])
    rb = r[...]
    cut = jnp.where(rb < CUTOFF,
                    0.5 * (jnp.cos(jnp.pi * rb / CUTOFF) + 1.0), 0.0)
    dkv_o[:, H:] = _silu(fb @ Wdv[...] + bdv[...]) * cut


def _edge_dense(f_ij, r_ij, Wdk, bdk, Wdv, bdv):
    grid = (E // EB,)
    be = pl.BlockSpec((EB, H), lambda i: (i, 0))
    b1 = pl.BlockSpec((EB, 1), lambda i: (i, 0))
    b2 = pl.BlockSpec((EB, 2 * H), lambda i: (i, 0))
    bw = lambda s: pl.BlockSpec(s, lambda i: (0, 0))
    return pl.pallas_call(
        _k2_body,
        grid=grid,
        in_specs=[be, b1, bw((H, H)), bw((1, H)), bw((H, H)), bw((1, H))],
        out_specs=b2,
        out_shape=jax.ShapeDtypeStruct((E, 2 * H), jnp.float32),
    )(f_ij, r_ij.reshape(E, 1), Wdk, bdk.reshape(1, H), Wdv, bdv.reshape(1, H))


def _sc_edge_a(qTT, kv, dkv, src, dst):
    CH = 40
    NW = 32
    EPW = E // NW
    PW = EPW // CH
    mesh = plsc.VectorSubcoreMesh(core_axis_name="c", subcore_axis_name="s")

    @functools.partial(
        pl.kernel, mesh=mesh,
        out_type=[jax.ShapeDtypeStruct((E, H), jnp.float32),
                  jax.ShapeDtypeStruct((E, H), jnp.float32)],
        scratch_types=[pltpu.VMEM((EPW,), jnp.int32),
                       pltpu.VMEM((EPW,), jnp.int32),
                       pltpu.VMEM((CH, 2 * H), jnp.float32),
                       pltpu.VMEM((CH, 2 * H), jnp.float32),
                       pltpu.VMEM((CH, 2 * H), jnp.float32),
                       pltpu.VMEM((CH, 2 * H), jnp.float32),
                       pltpu.VMEM((CH, 2 * H), jnp.float32),
                       pltpu.VMEM((CH, 2 * H), jnp.float32),
                       pltpu.VMEM((CH, H), jnp.float32),
                       pltpu.VMEM((CH, H), jnp.float32),
                       pltpu.SemaphoreType.DMA,
                       pltpu.SemaphoreType.DMA],
    )
    def k(qTT_h, kv_h, dkv_h, src_h, dst_h,
          ve_o, tte_o,
          dstb, srcb, qb0, qb1, kvb0, kvb1, db0, db1, veb, tteb,
          m0, m1):
        c = lax.axis_index("c")
        s = lax.axis_index("s")
        wid = c * 16 + s
        base_w = wid * EPW
        pltpu.sync_copy(dst_h.at[pl.ds(base_w, EPW)], dstb)
        pltpu.sync_copy(src_h.at[pl.ds(base_w, EPW)], srcb)
        qbufs = (qb0, qb1)
        kvbufs = (kvb0, kvb1)
        dbufs = (db0, db1)
        sems = (m0, m1)

        def issue(ci, b):
            off = ci * CH
            pltpu.async_copy(qTT_h.at[dstb.at[pl.ds(off, CH)]],
                             qbufs[b], sems[b])
            pltpu.async_copy(kv_h.at[srcb.at[pl.ds(off, CH)]],
                             kvbufs[b], sems[b])
            pltpu.async_copy(dkv_h.at[pl.ds(base_w + off, CH)],
                             dbufs[b], sems[b])

        def wait(b):
            pltpu.make_async_copy(qTT_h.at[dstb.at[pl.ds(0, CH)]],
                                  qbufs[b], sems[b]).wait()
            pltpu.make_async_copy(kv_h.at[srcb.at[pl.ds(0, CH)]],
                                  kvbufs[b], sems[b]).wait()
            pltpu.make_async_copy(dkv_h.at[pl.ds(0, CH)],
                                  dbufs[b], sems[b]).wait()

        issue(0, 0)

        def pair(cj, _):
            for b in range(2):
                ci = cj * 2 + b

                @pl.when(ci < PW)
                def _do(b=b, ci=ci):
                    @pl.when(ci + 1 < PW)
                    def _issue_next():
                        issue(ci + 1, 1 - b)
                    wait(b)
                    qb = qbufs[b]
                    kvb = kvbufs[b]
                    db = dbufs[b]

                    def edge(i, _):
                        for h in range(8):
                            sl = pl.ds(16 * h, 16)
                            slv = pl.ds(H + 16 * h, 16)
                            t = qb[i, sl] * kvb[i, sl] * db[i, sl]
                            sb = _lane_sum(t)
                            av = sb / (1.0 + jnp.exp(-sb))
                            veb[i, sl] = kvb[i, slv] * db[i, slv] * av
                            tteb[i, sl] = qb[i, slv]
                        return 0
                    lax.fori_loop(0, CH, edge, 0)
                    off = base_w + ci * CH
                    pltpu.sync_copy(veb, ve_o.at[pl.ds(off, CH)])
                    pltpu.sync_copy(tteb, tte_o.at[pl.ds(off, CH)])
            return 0
        lax.fori_loop(0, (PW + 1) // 2, pair, 0)

    return k(qTT, kv, dkv, src, dst)


def _k3_body(ve, d0, d1, d2, Ws, bs, s1_o, sd0_o, sd1_o, sd2_o, d48_o):
    s = _silu(ve[...] @ Ws[...] + bs[...])
    s1_o[...] = s[:, :H]
    s2 = s[:, H:]
    sd_out = (sd0_o, sd1_o, sd2_o)
    ds = (d0[...], d1[...], d2[...])
    for a in range(3):
        sd_out[a][...] = s2 * ds[a]
        d48_o[:, a * 16:(a + 1) * 16] = jnp.broadcast_to(ds[a], (EB, 16))


def _s_kernel(v_e, d_ij, Ws, bs):
    grid = (E // EB,)
    be = pl.BlockSpec((EB, H), lambda i: (i, 0))
    b1 = pl.BlockSpec((EB, 1), lambda i: (i, 0))
    bd = pl.BlockSpec((EB, 48), lambda i: (i, 0))
    bw = lambda s: pl.BlockSpec(s, lambda i: (0, 0))
    return pl.pallas_call(
        _k3_body,
        grid=grid,
        in_specs=[be, b1, b1, b1, bw((H, 2 * H)), bw((1, 2 * H))],
        out_specs=[be, be, be, be, bd],
        out_shape=[jax.ShapeDtypeStruct((E, H), jnp.float32)] * 4 +
                  [jax.ShapeDtypeStruct((E, 48), jnp.float32)],
    )(v_e, d_ij[:, 0:1], d_ij[:, 1:2], d_ij[:, 2:3], Ws,
      bs.reshape(1, 2 * H))


def _sc_msg_scatter(vec0, vec1, vec2, s1, sd0, sd1, sd2, v_e, src2d, dst2d):
    CH1 = 40
    NW = 32
    EPW = E // NW
    PW = EPW // CH1
    NPAD = 10240
    NT = NPAD // 16
    mesh = plsc.VectorSubcoreMesh(core_axis_name="c", subcore_axis_name="s")

    @functools.partial(
        pl.kernel, mesh=mesh,
        out_type=[jax.ShapeDtypeStruct((6 * N, H), jnp.float32),
                  jax.ShapeDtypeStruct((2 * N, H), jnp.float32)],
        scratch_types=[pltpu.VMEM((EPW,), jnp.int32),
                       pltpu.VMEM((EPW,), jnp.int32),
                       pltpu.VMEM((CH1,), jnp.int32),
                       pltpu.VMEM((CH1, H), jnp.float32),
                       pltpu.VMEM((CH1, H), jnp.float32),
                       pltpu.VMEM((CH1, H), jnp.float32),
                       pltpu.VMEM((CH1, H), jnp.float32),
                       pltpu.VMEM_SHARED((NPAD, H), jnp.float32),
                       pltpu.SemaphoreType.DMA,
                       pltpu.SemaphoreType.DMA,
                       pltpu.SemaphoreType.DMA],
    )
    def k(v0_h, v1_h, v2_h, s1_h, sd0_h, sd1_h, sd2_h, ve_h,
          src2_h, dst2_h,
          msum_o, xpart_o,
          srcb, dstb, dstc, vecb, s1b, sdb, msgb, vsh, m1, m2, m3):
        c = lax.axis_index("c")
        s = lax.axis_index("s")
        wid = c * 16 + s
        base_w = wid * EPW
        pltpu.sync_copy(src2_h.at[pl.ds(base_w, EPW)], srcb)
        pltpu.sync_copy(dst2_h.at[pl.ds(base_w, EPW)], dstb)

        def load_dstc(off):
            dstc[pl.ds(0, 16)] = dstb[pl.ds(off, 16)]
            dstc[pl.ds(16, 16)] = dstb[pl.ds(off + 16, 16)]
            dstc[pl.ds(24, 16)] = dstb[pl.ds(off + 24, 16)]
        vtabs = (v0_h, v1_h, v2_h)
        sdtabs = (sd0_h, sd1_h, sd2_h)
        zv = jnp.zeros((16,), jnp.float32)
        last = N - 15 * NT

        def zbuf(i, _):
            for g in range(8):
                msgb[i, pl.ds(g * 16, 16)] = zv
            return 0

        def zero_spmem():
            lax.fori_loop(0, CH1, zbuf, 0)
            for z in range(NT // CH1):
                pltpu.sync_copy(msgb, vsh.at[pl.ds(s * NT + z * CH1, CH1)])

        zero_spmem()
        plsc.subcore_barrier()

        for a in range(4):
            if a < 3:
                vtab = vtabs[a]
                sdtab = sdtabs[a]

                def chunk(ci, _, vtab=vtab, sdtab=sdtab):
                    off = ci * CH1
                    base = base_w + off
                    c1 = pltpu.async_copy(vtab.at[srcb.at[pl.ds(off, CH1)]],
                                          vecb, m1)
                    c2 = pltpu.async_copy(s1_h.at[pl.ds(base, CH1)], s1b, m2)
                    c3 = pltpu.async_copy(sdtab.at[pl.ds(base, CH1)], sdb, m3)
                    load_dstc(off)
                    c1.wait(); c2.wait(); c3.wait()

                    def edge(i, _):
                        for g8 in range(8):
                            sl = pl.ds(g8 * 16, 16)
                            msgb[i, sl] = (vecb[i, sl] * s1b[i, sl]
                                           + sdb[i, sl])
                        return 0
                    lax.fori_loop(0, CH1, edge, 0)
                    pltpu.sync_copy(msgb, vsh.at[dstc], add=True)
                    return 0
            else:
                def chunk(ci, _):
                    off = ci * CH1
                    base = base_w + off
                    pltpu.sync_copy(ve_h.at[pl.ds(base, CH1)], vecb)
                    load_dstc(off)
                    pltpu.sync_copy(vecb, vsh.at[dstc], add=True)
                    return 0
            lax.fori_loop(0, PW, chunk, 0)
            plsc.subcore_barrier()
            if a < 3:
                row0 = (c * 3 + a) * N
                out_ref = msum_o
            else:
                row0 = c * N
                out_ref = xpart_o

            @pl.when(s < 15)
            def _dump_full(row0=row0, out_ref=out_ref):
                pltpu.sync_copy(vsh.at[pl.ds(s * NT, NT)],
                                out_ref.at[pl.ds(row0 + s * NT, NT)])

            @pl.when(s == 15)
            def _dump_last(row0=row0, out_ref=out_ref):
                pltpu.sync_copy(vsh.at[pl.ds(15 * NT, last)],
                                out_ref.at[pl.ds(row0 + 15 * NT, last)])
            if a < 3:
                zero_spmem()
            plsc.subcore_barrier()

    return k(vec0, vec1, vec2, s1, sd0, sd1, sd2, v_e, src2d, dst2d)


def _sc_wdot(TabD, TabS, d48, src, dst):
    CH2 = 40
    NCHUNK = E // CH2
    NW = 32
    per_w = NCHUNK // NW
    mesh = plsc.VectorSubcoreMesh(core_axis_name="c", subcore_axis_name="s")

    @functools.partial(
        pl.kernel, mesh=mesh,
        out_type=[jax.ShapeDtypeStruct((E, H), jnp.float32)] * 3,
        scratch_types=[pltpu.VMEM((CH2,), jnp.int32),
                       pltpu.VMEM((CH2,), jnp.int32),
                       pltpu.VMEM((CH2, 6 * H), jnp.float32),
                       pltpu.VMEM((CH2, 6 * H), jnp.float32),
                       pltpu.VMEM((CH2, 48), jnp.float32),
                       pltpu.VMEM((CH2, H), jnp.float32),
                       pltpu.VMEM((CH2, H), jnp.float32),
                       pltpu.VMEM((CH2, H), jnp.float32),
                       pltpu.SemaphoreType.DMA,
                       pltpu.SemaphoreType.DMA,
                       pltpu.SemaphoreType.DMA],
    )
    def k(TabD_h, TabS_h, d48_h, src_h, dst_h,
          wt1_o, dvi_o, dvj_o,
          dsti, srci, Db, Sb, d48b, wt1b, dvib, dvjb, m1, m2, m3):
        c = lax.axis_index("c")
        s = lax.axis_index("s")
        wid = c * 16 + s

        def chunk(ci, _):
            g = wid + ci * NW
            base = g * CH2
            pltpu.sync_copy(dst_h.at[pl.ds(base, CH2)], dsti)
            pltpu.sync_copy(src_h.at[pl.ds(base, CH2)], srci)
            c1 = pltpu.async_copy(TabD_h.at[dsti], Db, m1)
            c2 = pltpu.async_copy(TabS_h.at[srci], Sb, m2)
            c3 = pltpu.async_copy(d48_h.at[pl.ds(base, CH2)], d48b, m3)
            c1.wait(); c2.wait(); c3.wait()

            def edge(i, _):
                d0v = d48b[i, pl.ds(0, 16)]
                d1v = d48b[i, pl.ds(16, 16)]
                d2v = d48b[i, pl.ds(32, 16)]
                for g8 in range(8):
                    o = g8 * 16
                    sl = pl.ds(o, 16)
                    vi0 = Db[i, pl.ds(o, 16)]
                    vi1 = Db[i, pl.ds(H + o, 16)]
                    vi2 = Db[i, pl.ds(2 * H + o, 16)]
                    A0 = Db[i, pl.ds(3 * H + o, 16)]
                    A1 = Db[i, pl.ds(4 * H + o, 16)]
                    A2 = Db[i, pl.ds(5 * H + o, 16)]
                    vj0 = Sb[i, pl.ds(o, 16)]
                    vj1 = Sb[i, pl.ds(H + o, 16)]
                    vj2 = Sb[i, pl.ds(2 * H + o, 16)]
                    B0 = Sb[i, pl.ds(3 * H + o, 16)]
                    B1 = Sb[i, pl.ds(4 * H + o, 16)]
                    B2 = Sb[i, pl.ds(5 * H + o, 16)]
                    wt1b[i, sl] = A0 * B0 + A1 * B1 + A2 * B2
                    dvib[i, sl] = d0v * vi0 + d1v * vi1 + d2v * vi2
                    dvjb[i, sl] = d0v * vj0 + d1v * vj1 + d2v * vj2
                return 0
            lax.fori_loop(0, CH2, edge, 0)
            pltpu.sync_copy(wt1b, wt1_o.at[pl.ds(base, CH2)])
            pltpu.sync_copy(dvib, dvi_o.at[pl.ds(base, CH2)])
            pltpu.sync_copy(dvjb, dvj_o.at[pl.ds(base, CH2)])
            return 0
        lax.fori_loop(0, per_w, chunk, 0)

    return k(TabD, TabS, d48, src, dst)


def _k4_body(f, wt1, dvi, dvj, tte, Wf, bf, Wwtrg, Wwsrc, Wttrg, Wtsrc,
             df_o):
    dvi_b = dvi[...]
    w_dot = wt1[...] - (dvi_b @ Wwtrg[...]) * (dvj[...] @ Wwsrc[...])
    t_dot = tte[...] - (dvi_b @ Wttrg[...]) * (dvi_b @ Wtsrc[...])
    ff = _silu(f[...] @ Wf[...] + bf[...])
    df_o[...] = ff[:, :H] * w_dot + ff[:, H:] * t_dot


def _df_kernel(f_ij, wt1, dvi, dvj, tte, Wf, bf, Wwtrg, Wwsrc, Wttrg, Wtsrc):
    grid = (E // EB,)
    be = pl.BlockSpec((EB, H), lambda i: (i, 0))
    bw = lambda s: pl.BlockSpec(s, lambda i: (0, 0))
    return pl.pallas_call(
        _k4_body,
        grid=grid,
        in_specs=[be, be, be, be, be,
                  bw((H, 2 * H)), bw((1, 2 * H)),
                  bw((H, H)), bw((H, H)), bw((H, H)), bw((H, H))],
        out_specs=be,
        out_shape=jax.ShapeDtypeStruct((E, H), jnp.float32),
    )(f_ij, wt1, dvi, dvj, tte, Wf, bf.reshape(1, 2 * H),
      Wwtrg, Wwsrc, Wttrg, Wtsrc)


def _k5_body(xp0, xp1, vdot, v30, v31, v32, ga0, gb0, ga1, gb1, ga2, gb2,
             Wo, bo, dx_o, dv0_o, dv1_o, dv2_o):
    o = (xp0[...] + xp1[...]) @ Wo[...] + bo[...]
    o1, o2, o3 = o[:, :H], o[:, H:2 * H], o[:, 2 * H:]
    dx_o[...] = vdot[...] * o2 + o3
    dv0_o[...] = v30[...] * o1 + ga0[...] + gb0[...]
    dv1_o[...] = v31[...] * o1 + ga1[...] + gb1[...]
    dv2_o[...] = v32[...] * o1 + ga2[...] + gb2[...]


def _node_epilogue(xpart, vec_dot, v3, msum, Wo, bo):
    grid = (N // NB,)
    nb = N // NB
    bn = pl.BlockSpec((NB, H), lambda i: (i, 0))
    bnR = pl.BlockSpec((NB, H), lambda i: (nb + i, 0))

    def bm(a, c):
        return pl.BlockSpec((NB, H),
                            lambda i, _a=a, _c=c: ((_c * 3 + _a) * nb + i, 0))
    bw = lambda s: pl.BlockSpec(s, lambda i: (0, 0))
    return pl.pallas_call(
        _k5_body,
        grid=grid,
        in_specs=[bn, bnR, bn, bn, bn, bn,
                  bm(0, 0), bm(0, 1), bm(1, 0), bm(1, 1), bm(2, 0), bm(2, 1),
                  bw((H, 3 * H)), bw((1, 3 * H))],
        out_specs=[bn] * 4,
        out_shape=[jax.ShapeDtypeStruct((N, H), jnp.float32)] * 4,
    )(xpart, xpart, vec_dot, v3[0], v3[1], v3[2],
      msum, msum, msum, msum, msum, msum,
      Wo, bo.reshape(1, 3 * H))


def kernel(x, vec, edge_index, r_ij, f_ij, d_ij, Wvec, Wq, bq, Wk, bk, Wv, bv,
           Wdk, bdk, Wdv, bdv, Ws, bs, Wf, bf, Wwsrc, Wwtrg, Wtsrc, Wttrg,
           Wo, bo):
    src, dst = edge_index[0], edge_index[1]
    (qTT, kv, vec_dot, A_cat, B_cat, v30, v31, v32) = \
        _node_precompute(x, vec, Wq, bq, Wk, bk, Wv, bv, Wvec, Wwtrg, Wwsrc,
                         Wttrg, Wtsrc)
    dkv = _edge_dense(f_ij, r_ij, Wdk, bdk, Wdv, bdv)

    v_e, TT_e = _sc_edge_a(qTT, kv, dkv, src, dst)

    s1, sd0, sd1, sd2, d48 = _s_kernel(v_e, d_ij, Ws, bs)

    vec_cat = vec.reshape(N, 3 * H)
    TabD = jnp.concatenate([vec_cat, A_cat], axis=1)
    TabS = jnp.concatenate([vec_cat, B_cat], axis=1)
    msum, xpart = _sc_msg_scatter(vec[:, 0, :], vec[:, 1, :], vec[:, 2, :],
                                  s1, sd0, sd1, sd2, v_e, src, dst)
    wt1, dvi, dvj = _sc_wdot(TabD, TabS, d48, src, dst)

    df = _df_kernel(f_ij, wt1, dvi, dvj, TT_e, Wf, bf, Wwtrg, Wwsrc,
                    Wttrg, Wtsrc)

    dx, dv0, dv1, dv2 = _node_epilogue(xpart, vec_dot, (v30, v31, v32),
                                       msum, Wo, bo)
    dvec = jnp.stack([dv0, dv1, dv2], axis=1)
    return (dx, dvec, df)


# speedup vs baseline: 7.1227x; 1.3233x over previous
import functools
import jax
import jax.numpy as jnp
from jax import lax
from jax.experimental import pallas as pl
from jax.experimental.pallas import tpu as pltpu
from jax.experimental.pallas import tpu_sc as plsc

N = 10000
E = 160000
H = 128
HEADS = 8
HD = H // HEADS
CUTOFF = 5.0

NB = 1000
EB = 1600


def _silu(x):
    return x * jax.nn.sigmoid(x)


_GDN = lax.GatherDimensionNumbers(offset_dims=(), collapsed_slice_dims=(0,),
                                  start_index_map=(0,))


def _lane_perm(x, perm):
    return lax.gather(x, perm[:, None], dimension_numbers=_GDN,
                      slice_sizes=(1,),
                      mode=lax.GatherScatterMode.PROMISE_IN_BOUNDS)


def _lane_sum(x):
    lanes = lax.iota(jnp.int32, 16)
    for stq in (1, 2, 4, 8):
        x = x + _lane_perm(x, lanes ^ stq)
    return x


def _k1_body(x, v0, v1, v2, Wq, bq, Wk, bk, Wv, bv, Wvec, Wwtrg, Wwsrc,
             Wttrg, Wtsrc,
             qTT_o, kv_o, vdot_o, A_o, B_o, v3a_o, v3b_o, v3c_o):
    xb = x[...]
    qTT_o[:, :H] = xb @ Wq[...] + bq[...]
    kv_o[:, :H] = xb @ Wk[...] + bk[...]
    kv_o[:, H:] = xb @ Wv[...] + bv[...]
    vs = (v0[...], v1[...], v2[...])
    Wv3 = Wvec[...]
    vdot = jnp.zeros_like(xb)
    TT = jnp.zeros_like(xb)
    Wt1 = Wttrg[...]
    Wt2 = Wtsrc[...]
    Ww1 = Wwtrg[...]
    Ww2 = Wwsrc[...]
    v3_out = (v3a_o, v3b_o, v3c_o)
    for a in range(3):
        va = vs[a]
        v123 = va @ Wv3
        vdot = vdot + v123[:, :H] * v123[:, H:2 * H]
        v3_out[a][...] = v123[:, 2 * H:]
        TT = TT + (va @ Wt1) * (va @ Wt2)
        A_o[:, a * H:(a + 1) * H] = va @ Ww1
        B_o[:, a * H:(a + 1) * H] = va @ Ww2
    vdot_o[...] = vdot
    qTT_o[:, H:] = TT


def _node_precompute(x, vec, Wq, bq, Wk, bk, Wv, bv, Wvec, Wwtrg, Wwsrc,
                     Wttrg, Wtsrc):
    grid = (N // NB,)
    bn = pl.BlockSpec((NB, H), lambda i: (i, 0))
    b2 = pl.BlockSpec((NB, 2 * H), lambda i: (i, 0))
    b3 = pl.BlockSpec((NB, 3 * H), lambda i: (i, 0))
    bw = lambda s: pl.BlockSpec(s, lambda i: (0, 0))
    outs = [jax.ShapeDtypeStruct((N, 2 * H), jnp.float32),
            jax.ShapeDtypeStruct((N, 2 * H), jnp.float32),
            jax.ShapeDtypeStruct((N, H), jnp.float32),
            jax.ShapeDtypeStruct((N, 3 * H), jnp.float32),
            jax.ShapeDtypeStruct((N, 3 * H), jnp.float32),
            jax.ShapeDtypeStruct((N, H), jnp.float32),
            jax.ShapeDtypeStruct((N, H), jnp.float32),
            jax.ShapeDtypeStruct((N, H), jnp.float32)]
    return pl.pallas_call(
        _k1_body,
        grid=grid,
        in_specs=[bn, bn, bn, bn,
                  bw((H, H)), bw((1, H)), bw((H, H)), bw((1, H)),
                  bw((H, H)), bw((1, H)), bw((H, 3 * H)),
                  bw((H, H)), bw((H, H)), bw((H, H)), bw((H, H))],
        out_specs=[b2, b2, bn, b3, b3, bn, bn, bn],
        out_shape=outs,
    )(x, vec[:, 0, :], vec[:, 1, :], vec[:, 2, :],
      Wq, bq.reshape(1, H), Wk, bk.reshape(1, H), Wv, bv.reshape(1, H),
      Wvec, Wwtrg, Wwsrc, Wttrg, Wtsrc)


def _k2_body(f, r, Wdk, bdk, Wdv, bdv, dkv_o):
    fb = f[...]
    dkv_o[:, :H] = _silu(fb @ Wdk[...] + bdk[...])
    rb = r[...]
    cut = jnp.where(rb < CUTOFF,
                    0.5 * (jnp.cos(jnp.pi * rb / CUTOFF) + 1.0), 0.0)
    dkv_o[:, H:] = _silu(fb @ Wdv[...] + bdv[...]) * cut


def _edge_dense(f_ij, r_ij, Wdk, bdk, Wdv, bdv):
    grid = (E // EB,)
    be = pl.BlockSpec((EB, H), lambda i: (i, 0))
    b1 = pl.BlockSpec((EB, 1), lambda i: (i, 0))
    b2 = pl.BlockSpec((EB, 2 * H), lambda i: (i, 0))
    bw = lambda s: pl.BlockSpec(s, lambda i: (0, 0))
    return pl.pallas_call(
        _k2_body,
        grid=grid,
        in_specs=[be, b1, bw((H, H)), bw((1, H)), bw((H, H)), bw((1, H))],
        out_specs=b2,
        out_shape=jax.ShapeDtypeStruct((E, 2 * H), jnp.float32),
    )(f_ij, r_ij.reshape(E, 1), Wdk, bdk.reshape(1, H), Wdv, bdv.reshape(1, H))


def _sc_edge_a(qTT, kv, src, dst):
    CH = 80
    NW = 32
    EPW = E // NW
    PW = EPW // CH
    mesh = plsc.VectorSubcoreMesh(core_axis_name="c", subcore_axis_name="s")

    @functools.partial(
        pl.kernel, mesh=mesh,
        out_type=[jax.ShapeDtypeStruct((E, 2 * H), jnp.float32),
                  jax.ShapeDtypeStruct((E, 2 * H), jnp.float32)],
        scratch_types=[pltpu.VMEM((EPW,), jnp.int32),
                       pltpu.VMEM((EPW,), jnp.int32),
                       pltpu.VMEM((CH, 2 * H), jnp.float32),
                       pltpu.VMEM((CH, 2 * H), jnp.float32),
                       pltpu.VMEM((CH, 2 * H), jnp.float32),
                       pltpu.VMEM((CH, 2 * H), jnp.float32),
                       pltpu.SemaphoreType.DMA,
                       pltpu.SemaphoreType.DMA,
                       pltpu.SemaphoreType.DMA,
                       pltpu.SemaphoreType.DMA],
    )
    def k(qTT_h, kv_h, src_h, dst_h,
          qe_o, ke_o,
          dstb, srcb, qb0, qb1, kvb0, kvb1,
          m0, m1, w0, w1):
        c = lax.axis_index("c")
        s = lax.axis_index("s")
        wid = c * 16 + s
        base_w = wid * EPW
        pltpu.sync_copy(dst_h.at[pl.ds(base_w, EPW)], dstb)
        pltpu.sync_copy(src_h.at[pl.ds(base_w, EPW)], srcb)
        qbufs = (qb0, qb1)
        kvbufs = (kvb0, kvb1)
        sems = (m0, m1)
        wsems = (w0, w1)

        def issue(ci, b):
            off = ci * CH
            pltpu.async_copy(qTT_h.at[dstb.at[pl.ds(off, CH)]],
                             qbufs[b], sems[b])
            pltpu.async_copy(kv_h.at[srcb.at[pl.ds(off, CH)]],
                             kvbufs[b], sems[b])

        def wait_gather(b):
            pltpu.make_async_copy(qTT_h.at[dstb.at[pl.ds(0, CH)]],
                                  qbufs[b], sems[b]).wait()
            pltpu.make_async_copy(kv_h.at[srcb.at[pl.ds(0, CH)]],
                                  kvbufs[b], sems[b]).wait()

        def wait_write(b):
            pltpu.make_async_copy(qbufs[b], qe_o.at[pl.ds(0, CH)],
                                  wsems[b]).wait()
            pltpu.make_async_copy(kvbufs[b], ke_o.at[pl.ds(0, CH)],
                                  wsems[b]).wait()

        issue(0, 0)

        def pair(cj, _):
            for b in range(2):
                ci = cj * 2 + b

                @pl.when(ci < PW)
                def _do(b=b, ci=ci):
                    @pl.when(ci >= 2)
                    def _drain_prev_write():
                        wait_write(b)

                    @pl.when(ci + 1 < PW)
                    def _issue_next():
                        issue(ci + 1, 1 - b)
                    wait_gather(b)
                    off = base_w + ci * CH
                    pltpu.async_copy(qbufs[b], qe_o.at[pl.ds(off, CH)],
                                     wsems[b])
                    pltpu.async_copy(kvbufs[b], ke_o.at[pl.ds(off, CH)],
                                     wsems[b])
            return 0
        lax.fori_loop(0, (PW + 1) // 2, pair, 0)
        wait_write(0)
        wait_write(1)

    return k(qTT, kv, src, dst)


def _k3_body(qe, ke, dkv, d0, d1, d2, hsum, hbc, Ws, bs,
             ve_o, s1_o, sd0_o, sd1_o, sd2_o, d48_o):
    q_i = qe[:, :H]
    k_j = ke[:, :H]
    v_j = ke[:, H:]
    dk = dkv[:, :H]
    dvc = dkv[:, H:]
    qkd = q_i * k_j * dk
    attn = _silu(qkd @ hsum[...])
    v_e = v_j * dvc * (attn @ hbc[...])
    ve_o[...] = v_e
    s = _silu(v_e @ Ws[...] + bs[...])
    s1_o[...] = s[:, :H]
    s2 = s[:, H:]
    sd_out = (sd0_o, sd1_o, sd2_o)
    ds = (d0[...], d1[...], d2[...])
    for a in range(3):
        sd_out[a][...] = s2 * ds[a]
        d48_o[:, a * 16:(a + 1) * 16] = jnp.broadcast_to(ds[a], (EB, 16))


def _s_kernel(qe, ke, dkv, d_ij, Ws, bs):
    grid = (E // EB,)
    be = pl.BlockSpec((EB, H), lambda i: (i, 0))
    b2 = pl.BlockSpec((EB, 2 * H), lambda i: (i, 0))
    b1 = pl.BlockSpec((EB, 1), lambda i: (i, 0))
    bd = pl.BlockSpec((EB, 48), lambda i: (i, 0))
    bw = lambda s: pl.BlockSpec(s, lambda i: (0, 0))
    hsum = (jnp.arange(H)[:, None] // HD == jnp.arange(HEADS)[None, :]
            ).astype(jnp.float32)
    return pl.pallas_call(
        _k3_body,
        grid=grid,
        in_specs=[b2, b2, b2, b1, b1, b1,
                  bw((H, HEADS)), bw((HEADS, H)),
                  bw((H, 2 * H)), bw((1, 2 * H))],
        out_specs=[be, be, be, be, be, bd],
        out_shape=[jax.ShapeDtypeStruct((E, H), jnp.float32)] * 5 +
                  [jax.ShapeDtypeStruct((E, 48), jnp.float32)],
    )(qe, ke, dkv, d_ij[:, 0:1], d_ij[:, 1:2], d_ij[:, 2:3],
      hsum, hsum.T, Ws, bs.reshape(1, 2 * H))


def _sc_msg_scatter(vec0, vec1, vec2, s1, sd0, sd1, sd2, v_e, src2d, dst2d):
    CH1 = 40
    NW = 32
    EPW = E // NW
    PW = EPW // CH1
    NPAD = 10240
    NT = NPAD // 16
    mesh = plsc.VectorSubcoreMesh(core_axis_name="c", subcore_axis_name="s")

    @functools.partial(
        pl.kernel, mesh=mesh,
        out_type=[jax.ShapeDtypeStruct((6 * N, H), jnp.float32),
                  jax.ShapeDtypeStruct((2 * N, H), jnp.float32)],
        scratch_types=[pltpu.VMEM((EPW,), jnp.int32),
                       pltpu.VMEM((EPW,), jnp.int32),
                       pltpu.VMEM((CH1,), jnp.int32),
                       pltpu.VMEM((CH1, H), jnp.float32),
                       pltpu.VMEM((CH1, H), jnp.float32),
                       pltpu.VMEM((CH1, H), jnp.float32),
                       pltpu.VMEM((CH1, H), jnp.float32),
                       pltpu.VMEM_SHARED((NPAD, H), jnp.float32),
                       pltpu.SemaphoreType.DMA,
                       pltpu.SemaphoreType.DMA,
                       pltpu.SemaphoreType.DMA],
    )
    def k(v0_h, v1_h, v2_h, s1_h, sd0_h, sd1_h, sd2_h, ve_h,
          src2_h, dst2_h,
          msum_o, xpart_o,
          srcb, dstb, dstc, vecb, s1b, sdb, msgb, vsh, m1, m2, m3):
        c = lax.axis_index("c")
        s = lax.axis_index("s")
        wid = c * 16 + s
        base_w = wid * EPW
        pltpu.sync_copy(src2_h.at[pl.ds(base_w, EPW)], srcb)
        pltpu.sync_copy(dst2_h.at[pl.ds(base_w, EPW)], dstb)

        def load_dstc(off):
            dstc[pl.ds(0, 16)] = dstb[pl.ds(off, 16)]
            dstc[pl.ds(16, 16)] = dstb[pl.ds(off + 16, 16)]
            dstc[pl.ds(24, 16)] = dstb[pl.ds(off + 24, 16)]
        vtabs = (v0_h, v1_h, v2_h)
        sdtabs = (sd0_h, sd1_h, sd2_h)
        zv = jnp.zeros((16,), jnp.float32)
        last = N - 15 * NT

        def zbuf(i, _):
            for g in range(8):
                msgb[i, pl.ds(g * 16, 16)] = zv
            return 0

        def zero_spmem():
            lax.fori_loop(0, CH1, zbuf, 0)
            for z in range(NT // CH1):
                pltpu.sync_copy(msgb, vsh.at[pl.ds(s * NT + z * CH1, CH1)])

        zero_spmem()
        plsc.subcore_barrier()

        for a in range(4):
            if a < 3:
                vtab = vtabs[a]
                sdtab = sdtabs[a]

                def chunk(ci, _, vtab=vtab, sdtab=sdtab):
                    off = ci * CH1
                    base = base_w + off
                    c1 = pltpu.async_copy(vtab.at[srcb.at[pl.ds(off, CH1)]],
                                          vecb, m1)
                    c2 = pltpu.async_copy(s1_h.at[pl.ds(base, CH1)], s1b, m2)
                    c3 = pltpu.async_copy(sdtab.at[pl.ds(base, CH1)], sdb, m3)
                    load_dstc(off)
                    c1.wait(); c2.wait(); c3.wait()

                    def edge(i, _):
                        for g8 in range(8):
                            sl = pl.ds(g8 * 16, 16)
                            msgb[i, sl] = (vecb[i, sl] * s1b[i, sl]
                                           + sdb[i, sl])
                        return 0
                    lax.fori_loop(0, CH1, edge, 0)
                    pltpu.sync_copy(msgb, vsh.at[dstc], add=True)
                    return 0
            else:
                def chunk(ci, _):
                    off = ci * CH1
                    base = base_w + off
                    pltpu.sync_copy(ve_h.at[pl.ds(base, CH1)], vecb)
                    load_dstc(off)
                    pltpu.sync_copy(vecb, vsh.at[dstc], add=True)
                    return 0
            lax.fori_loop(0, PW, chunk, 0)
            plsc.subcore_barrier()
            if a < 3:
                row0 = (c * 3 + a) * N
                out_ref = msum_o
            else:
                row0 = c * N
                out_ref = xpart_o

            @pl.when(s < 15)
            def _dump_full(row0=row0, out_ref=out_ref):
                pltpu.sync_copy(vsh.at[pl.ds(s * NT, NT)],
                                out_ref.at[pl.ds(row0 + s * NT, NT)])

            @pl.when(s == 15)
            def _dump_last(row0=row0, out_ref=out_ref):
                pltpu.sync_copy(vsh.at[pl.ds(15 * NT, last)],
                                out_ref.at[pl.ds(row0 + 15 * NT, last)])
            if a < 3:
                zero_spmem()
            plsc.subcore_barrier()

    return k(vec0, vec1, vec2, s1, sd0, sd1, sd2, v_e, src2d, dst2d)


def _sc_wdot(TabD, TabS, d48, src, dst):
    CH2 = 40
    NCHUNK = E // CH2
    NW = 32
    per_w = NCHUNK // NW
    mesh = plsc.VectorSubcoreMesh(core_axis_name="c", subcore_axis_name="s")

    @functools.partial(
        pl.kernel, mesh=mesh,
        out_type=[jax.ShapeDtypeStruct((E, H), jnp.float32)] * 3,
        scratch_types=[pltpu.VMEM((CH2,), jnp.int32),
                       pltpu.VMEM((CH2,), jnp.int32),
                       pltpu.VMEM((CH2, 6 * H), jnp.float32),
                       pltpu.VMEM((CH2, 6 * H), jnp.float32),
                       pltpu.VMEM((CH2, 48), jnp.float32),
                       pltpu.VMEM((CH2, H), jnp.float32),
                       pltpu.VMEM((CH2, H), jnp.float32),
                       pltpu.VMEM((CH2, H), jnp.float32),
                       pltpu.SemaphoreType.DMA,
                       pltpu.SemaphoreType.DMA,
                       pltpu.SemaphoreType.DMA],
    )
    def k(TabD_h, TabS_h, d48_h, src_h, dst_h,
          wt1_o, dvi_o, dvj_o,
          dsti, srci, Db, Sb, d48b, wt1b, dvib, dvjb, m1, m2, m3):
        c = lax.axis_index("c")
        s = lax.axis_index("s")
        wid = c * 16 + s

        def chunk(ci, _):
            g = wid + ci * NW
            base = g * CH2
            pltpu.sync_copy(dst_h.at[pl.ds(base, CH2)], dsti)
            pltpu.sync_copy(src_h.at[pl.ds(base, CH2)], srci)
            c1 = pltpu.async_copy(TabD_h.at[dsti], Db, m1)
            c2 = pltpu.async_copy(TabS_h.at[srci], Sb, m2)
            c3 = pltpu.async_copy(d48_h.at[pl.ds(base, CH2)], d48b, m3)
            c1.wait(); c2.wait(); c3.wait()

            def edge(i, _):
                d0v = d48b[i, pl.ds(0, 16)]
                d1v = d48b[i, pl.ds(16, 16)]
                d2v = d48b[i, pl.ds(32, 16)]
                for g8 in range(8):
                    o = g8 * 16
                    sl = pl.ds(o, 16)
                    vi0 = Db[i, pl.ds(o, 16)]
                    vi1 = Db[i, pl.ds(H + o, 16)]
                    vi2 = Db[i, pl.ds(2 * H + o, 16)]
                    A0 = Db[i, pl.ds(3 * H + o, 16)]
                    A1 = Db[i, pl.ds(4 * H + o, 16)]
                    A2 = Db[i, pl.ds(5 * H + o, 16)]
                    vj0 = Sb[i, pl.ds(o, 16)]
                    vj1 = Sb[i, pl.ds(H + o, 16)]
                    vj2 = Sb[i, pl.ds(2 * H + o, 16)]
                    B0 = Sb[i, pl.ds(3 * H + o, 16)]
                    B1 = Sb[i, pl.ds(4 * H + o, 16)]
                    B2 = Sb[i, pl.ds(5 * H + o, 16)]
                    wt1b[i, sl] = A0 * B0 + A1 * B1 + A2 * B2
                    dvib[i, sl] = d0v * vi0 + d1v * vi1 + d2v * vi2
                    dvjb[i, sl] = d0v * vj0 + d1v * vj1 + d2v * vj2
                return 0
            lax.fori_loop(0, CH2, edge, 0)
            pltpu.sync_copy(wt1b, wt1_o.at[pl.ds(base, CH2)])
            pltpu.sync_copy(dvib, dvi_o.at[pl.ds(base, CH2)])
            pltpu.sync_copy(dvjb, dvj_o.at[pl.ds(base, CH2)])
            return 0
        lax.fori_loop(0, per_w, chunk, 0)

    return k(TabD, TabS, d48, src, dst)


def _k4_body(f, wt1, dvi, dvj, qe, Wf, bf, Wwtrg, Wwsrc, Wttrg, Wtsrc,
             df_o):
    dvi_b = dvi[...]
    w_dot = wt1[...] - (dvi_b @ Wwtrg[...]) * (dvj[...] @ Wwsrc[...])
    t_dot = qe[:, H:] - (dvi_b @ Wttrg[...]) * (dvi_b @ Wtsrc[...])
    ff = _silu(f[...] @ Wf[...] + bf[...])
    df_o[...] = ff[:, :H] * w_dot + ff[:, H:] * t_dot


def _df_kernel(f_ij, wt1, dvi, dvj, qe, Wf, bf, Wwtrg, Wwsrc, Wttrg, Wtsrc):
    grid = (E // EB,)
    be = pl.BlockSpec((EB, H), lambda i: (i, 0))
    b2 = pl.BlockSpec((EB, 2 * H), lambda i: (i, 0))
    bw = lambda s: pl.BlockSpec(s, lambda i: (0, 0))
    return pl.pallas_call(
        _k4_body,
        grid=grid,
        in_specs=[be, be, be, be, b2,
                  bw((H, 2 * H)), bw((1, 2 * H)),
                  bw((H, H)), bw((H, H)), bw((H, H)), bw((H, H))],
        out_specs=be,
        out_shape=jax.ShapeDtypeStruct((E, H), jnp.float32),
    )(f_ij, wt1, dvi, dvj, qe, Wf, bf.reshape(1, 2 * H),
      Wwtrg, Wwsrc, Wttrg, Wtsrc)


def _k5_body(xp0, xp1, vdot, v30, v31, v32, ga0, gb0, ga1, gb1, ga2, gb2,
             Wo, bo, dx_o, dv0_o, dv1_o, dv2_o):
    o = (xp0[...] + xp1[...]) @ Wo[...] + bo[...]
    o1, o2, o3 = o[:, :H], o[:, H:2 * H], o[:, 2 * H:]
    dx_o[...] = vdot[...] * o2 + o3
    dv0_o[...] = v30[...] * o1 + ga0[...] + gb0[...]
    dv1_o[...] = v31[...] * o1 + ga1[...] + gb1[...]
    dv2_o[...] = v32[...] * o1 + ga2[...] + gb2[...]


def _node_epilogue(xpart, vec_dot, v3, msum, Wo, bo):
    grid = (N // NB,)
    nb = N // NB
    bn = pl.BlockSpec((NB, H), lambda i: (i, 0))
    bnR = pl.BlockSpec((NB, H), lambda i: (nb + i, 0))

    def bm(a, c):
        return pl.BlockSpec((NB, H),
                            lambda i, _a=a, _c=c: ((_c * 3 + _a) * nb + i, 0))
    bw = lambda s: pl.BlockSpec(s, lambda i: (0, 0))
    return pl.pallas_call(
        _k5_body,
        grid=grid,
        in_specs=[bn, bnR, bn, bn, bn, bn,
                  bm(0, 0), bm(0, 1), bm(1, 0), bm(1, 1), bm(2, 0), bm(2, 1),
                  bw((H, 3 * H)), bw((1, 3 * H))],
        out_specs=[bn] * 4,
        out_shape=[jax.ShapeDtypeStruct((N, H), jnp.float32)] * 4,
    )(xpart, xpart, vec_dot, v3[0], v3[1], v3[2],
      msum, msum, msum, msum, msum, msum,
      Wo, bo.reshape(1, 3 * H))


def kernel(x, vec, edge_index, r_ij, f_ij, d_ij, Wvec, Wq, bq, Wk, bk, Wv, bv,
           Wdk, bdk, Wdv, bdv, Ws, bs, Wf, bf, Wwsrc, Wwtrg, Wtsrc, Wttrg,
           Wo, bo):
    src, dst = edge_index[0], edge_index[1]
    (qTT, kv, vec_dot, A_cat, B_cat, v30, v31, v32) = \
        _node_precompute(x, vec, Wq, bq, Wk, bk, Wv, bv, Wvec, Wwtrg, Wwsrc,
                         Wttrg, Wtsrc)
    dkv = _edge_dense(f_ij, r_ij, Wdk, bdk, Wdv, bdv)

    qe, ke = _sc_edge_a(qTT, kv, src, dst)

    v_e, s1, sd0, sd1, sd2, d48 = _s_kernel(qe, ke, dkv, d_ij, Ws, bs)

    vec_cat = vec.reshape(N, 3 * H)
    TabD = jnp.concatenate([vec_cat, A_cat], axis=1)
    TabS = jnp.concatenate([vec_cat, B_cat], axis=1)
    msum, xpart = _sc_msg_scatter(vec[:, 0, :], vec[:, 1, :], vec[:, 2, :],
                                  s1, sd0, sd1, sd2, v_e, src, dst)
    wt1, dvi, dvj = _sc_wdot(TabD, TabS, d48, src, dst)

    df = _df_kernel(f_ij, wt1, dvi, dvj, qe, Wf, bf, Wwtrg, Wwsrc,
                    Wttrg, Wtsrc)

    dx, dv0, dv1, dv2 = _node_epilogue(xpart, vec_dot, (v30, v31, v32),
                                       msum, Wo, bo)
    dvec = jnp.stack([dv0, dv1, dv2], axis=1)
    return (dx, dvec, df)
